# baseline (speedup 1.0000x reference)
import jax
import jax.numpy as jnp
from jax.experimental import pallas as pl

N = 10000
E = 160000
IN_DIM = 128
HID = 64
HEADS = 6
HC = HID * HEADS
OUT_DIM = 32
NUM_GRAPHS = 64


def _gatv2(h, src, dst, ea, p):
    n = h.shape[0]
    cnt = jax.ops.segment_sum(jnp.ones((ea.shape[0], 1), jnp.float32), dst, n)
    loop_attr = jax.ops.segment_sum(ea, dst, n) / jnp.maximum(cnt, 1.0)
    ar = jnp.arange(n)
    src2 = jnp.concatenate([src, ar])
    dst2 = jnp.concatenate([dst, ar])
    ea2 = jnp.concatenate([ea, loop_attr], axis=0)
    xl = (h @ p["Wl"] + p["bl"]).reshape(n, HEADS, HID)
    xr = (h @ p["Wr"] + p["br"]).reshape(n, HEADS, HID)
    eemb = (ea2 @ p["We"]).reshape(-1, HEADS, HID)
    e = jax.nn.leaky_relu(xl[src2] + xr[dst2] + eemb, 0.2)
    logits = jnp.sum(e * p["att"][None], axis=-1)
    m = jax.ops.segment_max(logits, dst2, n)
    a = jnp.exp(logits - m[dst2])
    den = jax.ops.segment_sum(a, dst2, n)
    a = a / den[dst2]
    out = jax.ops.segment_sum(a[..., None] * xl[src2], dst2, n)
    return out.reshape(n, HC) + p["bias"]


def _tanh_pallas(x):
    def body(x_ref, o_ref):
        o_ref[...] = jnp.tanh(x_ref[...])
    return pl.pallas_call(body, out_shape=jax.ShapeDtypeStruct(x.shape, x.dtype))(x)


def kernel(x, edge_index, edge_attr, node_type_mask, batch, params):
    n = x.shape[0]
    h = jnp.zeros((n, HID), jnp.float32)
    for t, name in enumerate(["joint", "obj", "tcp", "goal"]):
        proj = x @ params["W_" + name] + params["b_" + name]
        h = jnp.where((node_type_mask == t)[:, None], proj, h)
    h = jax.nn.relu(h)
    src, dst = edge_index[0], edge_index[1]
    for p in params["convs"]:
        h = jax.nn.relu(_gatv2(h, src, dst, edge_attr, p))
    cnt = jax.ops.segment_sum(jnp.ones((n,), jnp.float32), batch, NUM_GRAPHS)
    pooled = jax.ops.segment_sum(h, batch, NUM_GRAPHS) / jnp.maximum(cnt, 1.0)[:, None]
    return _tanh_pallas(pooled @ params["W_out"] + params["b_out"])


# speedup vs baseline: 5.9409x; 5.9409x over previous
import functools

import jax
import jax.numpy as jnp
from jax import lax
from jax.experimental import pallas as pl
from jax.experimental.pallas import tpu as pltpu
from jax.experimental.pallas import tpu_sc as plsc

N = 10000
E = 160000
IN_DIM = 128
HID = 64
HEADS = 6
HC = HID * HEADS
OUT_DIM = 32
NUM_GRAPHS = 64

H2 = HC // 2
AW = H2 + 16
R = 1000
NB = N // R
EB = 2000
NEB = E // EB
TPS = N // 16
ZR = 125
B = 80
BE = 16
EPT = E // 16
NBAT = EPT // B
B0 = 40

_f32 = jnp.float32


def _halves(W):
    return W.reshape(W.shape[0], 2, H2).transpose(1, 0, 2)


def _sc_loopattr_body(ea_hbm, dst_hbm, z16_hbm, out_hbm,
                      la_sh, ea_v, dst_v, zer_v, sem):
    c = lax.axis_index("c")
    s = lax.axis_index("s")
    pltpu.sync_copy(z16_hbm, zer_v)
    for r in range(TPS // ZR):
        pltpu.sync_copy(zer_v, la_sh.at[pl.ds(s * TPS + r * ZR, ZR)])
    plsc.subcore_barrier()
    wid = s * 2 + c
    ebase = wid * (E // 32)

    def batch(b, carry):
        base = ebase + b * B0
        pltpu.sync_copy(dst_hbm.at[pl.ds(base, B0)], dst_v)
        pltpu.sync_copy(ea_hbm.at[pl.ds(base, B0)], ea_v)
        pltpu.sync_copy(ea_v, la_sh.at[dst_v], add=True)
        return carry

    lax.fori_loop(0, (E // 32) // B0, batch, 0)
    plsc.subcore_barrier()
    for r in range(TPS // ZR):
        rows = s * TPS + r * ZR
        pltpu.sync_copy(la_sh.at[pl.ds(rows, ZR)], zer_v)
        pltpu.sync_copy(zer_v, out_hbm.at[pl.ds(c * N + rows, ZR)])


def _sc_loopattr(ea_pad, dst, z16):
    mesh = plsc.VectorSubcoreMesh(core_axis_name="c", subcore_axis_name="s")
    return pl.kernel(
        _sc_loopattr_body,
        out_type=jax.ShapeDtypeStruct((2 * N, 16), _f32),
        mesh=mesh,
        compiler_params=pltpu.CompilerParams(needs_layout_passes=False, use_tc_tiling_on_sc=False),
        scratch_types=[
            pltpu.VMEM_SHARED((N, 16), _f32),
            pltpu.VMEM((B0, 16), _f32),
            pltpu.VMEM((B0,), jnp.int32),
            pltpu.VMEM((ZR, 16), _f32),
            pltpu.SemaphoreType.DMA,
        ],
    )(ea_pad, dst, z16)


def _sc_edge_body(xl_hbm, xr_hbm, em_hbm, src_hbm, dst_hbm, att_hbm, z192_hbm,
                  out_hbm, w_hbm,
                  acc_sh, src_v, dst_v, srcg_v, dstg_v,
                  xl_v, xr_v, em_v, w_v, att_v, red_v, sem):
    c = lax.axis_index("c")
    s = lax.axis_index("s")
    cN = c * N
    pltpu.sync_copy(z192_hbm, xl_v)
    for r in range(TPS // BE):
        pltpu.sync_copy(xl_v, acc_sh.at[pl.ds(s * TPS + r * BE, BE)])
    pltpu.sync_copy(xl_v.at[pl.ds(0, TPS % BE)],
                    acc_sh.at[pl.ds(s * TPS + (TPS // BE) * BE, TPS % BE)])
    pltpu.sync_copy(att_hbm.at[c, 0], att_v)
    plsc.subcore_barrier()
    ebase = s * EPT
    iota = lax.iota(jnp.int32, 16)

    def batch(b, carry):
        base = ebase + b * BE
        pltpu.sync_copy(src_hbm.at[pl.ds(base, BE)], src_v)
        pltpu.sync_copy(dst_hbm.at[pl.ds(base, BE)], dst_v)
        srcg_v[pl.ds(0, 16)] = src_v[pl.ds(0, 16)] + cN
        dstg_v[pl.ds(0, 16)] = dst_v[pl.ds(0, 16)] + cN
        pltpu.async_copy(xl_hbm.at[srcg_v], xl_v, sem).wait()
        pltpu.async_copy(xr_hbm.at[dstg_v], xr_v, sem).wait()
        pltpu.sync_copy(em_hbm.at[pl.ds(c * E + base, BE)], em_v)

        def edge(i, carry2):
            irow = iota * 0 + i
            ws = []
            for u in range(3):
                acc = None
                for q in range(4):
                    j = u * 4 + q
                    t = (xl_v[i, pl.ds(j * 16, 16)]
                         + xr_v[i, pl.ds(j * 16, 16)]
                         + em_v[i, pl.ds(j * 16, 16)])
                    t = jnp.maximum(t, 0.2 * t)
                    t = t * att_v[pl.ds(j * 16, 16)]
                    acc = t if acc is None else acc + t
                for k in (8, 4, 2, 1):
                    red_v[pl.ds(u * 16, 16)] = acc
                    acc = acc + plsc.load_gather(
                        red_v, [u * 16 + jnp.bitwise_xor(iota, k)])
                lself_u = plsc.load_gather(
                    xr_v, [irow, jnp.full((16,), H2 + u, jnp.int32)])
                ws.append(jnp.exp(acc - lself_u))
            w_v[i, pl.ds(0, 16)] = jnp.where(
                iota == 0, ws[0], jnp.where(iota == 1, ws[1],
                                            jnp.where(iota == 2, ws[2], 0.0)))
            for u in range(3):
                for q in range(4):
                    j = u * 4 + q
                    xl_v[i, pl.ds(j * 16, 16)] = (
                        xl_v[i, pl.ds(j * 16, 16)] * ws[u])
            return carry2

        lax.fori_loop(0, BE, edge, 0)
        pltpu.sync_copy(xl_v, acc_sh.at[dst_v], add=True)
        pltpu.sync_copy(w_v, w_hbm.at[pl.ds(c * E + base, BE)])
        return carry

    lax.fori_loop(0, EPT // BE, batch, 0)
    plsc.subcore_barrier()
    for r in range(TPS // BE):
        rows = s * TPS + r * BE
        pltpu.sync_copy(acc_sh.at[pl.ds(rows, BE)], xl_v)
        pltpu.sync_copy(xl_v, out_hbm.at[pl.ds(cN + rows, BE)])
    rows = s * TPS + (TPS // BE) * BE
    pltpu.sync_copy(acc_sh.at[pl.ds(rows, TPS % BE)], xl_v.at[pl.ds(0, TPS % BE)])
    pltpu.sync_copy(xl_v.at[pl.ds(0, TPS % BE)], out_hbm.at[pl.ds(cN + rows, TPS % BE)])


def _sc_edge(xl, xr_aug, eemb, src, dst, att2, z192):
    mesh = plsc.VectorSubcoreMesh(core_axis_name="c", subcore_axis_name="s")
    return pl.kernel(
        _sc_edge_body,
        out_type=[jax.ShapeDtypeStruct((2 * N, H2), _f32),
                  jax.ShapeDtypeStruct((2 * E, 16), _f32)],
        mesh=mesh,
        compiler_params=pltpu.CompilerParams(needs_layout_passes=False, use_tc_tiling_on_sc=False),
        scratch_types=[
            pltpu.VMEM_SHARED((N, H2), _f32),
            pltpu.VMEM((BE,), jnp.int32),
            pltpu.VMEM((BE,), jnp.int32),
            pltpu.VMEM((BE,), jnp.int32),
            pltpu.VMEM((BE,), jnp.int32),
            pltpu.VMEM((BE, H2), _f32),
            pltpu.VMEM((BE, AW), _f32),
            pltpu.VMEM((BE, H2), _f32),
            pltpu.VMEM((BE, 16), _f32),
            pltpu.VMEM((H2,), _f32),
            pltpu.VMEM((48,), _f32),
            pltpu.SemaphoreType.DMA,
        ],
    )(xl, xr_aug, eemb, src, dst, att2, z192)


def _sc_den_body(w_hbm, dst_hbm, z16_hbm, out_hbm,
                 den_sh, w_v, dst_v, zer_v, sem):
    c = lax.axis_index("c")
    s = lax.axis_index("s")
    pltpu.sync_copy(z16_hbm, zer_v)
    for r in range(TPS // ZR):
        pltpu.sync_copy(zer_v, den_sh.at[pl.ds(s * TPS + r * ZR, ZR)])
    plsc.subcore_barrier()
    ebase = s * EPT

    def batch(b, carry):
        base = ebase + b * B
        pltpu.sync_copy(dst_hbm.at[pl.ds(base, B)], dst_v)
        pltpu.sync_copy(w_hbm.at[pl.ds(c * E + base, B)], w_v)
        pltpu.sync_copy(w_v, den_sh.at[dst_v], add=True)
        return carry

    lax.fori_loop(0, EPT // B, batch, 0)
    plsc.subcore_barrier()
    for r in range(TPS // ZR):
        rows = s * TPS + r * ZR
        pltpu.sync_copy(den_sh.at[pl.ds(rows, ZR)], zer_v)
        pltpu.sync_copy(zer_v, out_hbm.at[pl.ds(c * N + rows, ZR)])


def _sc_den(w, dst, z16):
    mesh = plsc.VectorSubcoreMesh(core_axis_name="c", subcore_axis_name="s")
    return pl.kernel(
        _sc_den_body,
        out_type=jax.ShapeDtypeStruct((2 * N, 16), _f32),
        mesh=mesh,
        compiler_params=pltpu.CompilerParams(needs_layout_passes=False, use_tc_tiling_on_sc=False),
        scratch_types=[
            pltpu.VMEM_SHARED((N, 16), _f32),
            pltpu.VMEM((B, 16), _f32),
            pltpu.VMEM((B,), jnp.int32),
            pltpu.VMEM((ZR, 16), _f32),
            pltpu.SemaphoreType.DMA,
        ],
    )(w, dst, z16)


def _lself_cols(xl, xr, el, att_row):
    t = xl + xr + el
    t = jnp.maximum(t, 0.2 * t)
    ta = t * att_row[None, :]
    ls = [jnp.sum(ta[:, u * HID:(u + 1) * HID], axis=1, keepdims=True)
          for u in range(3)]
    pad = jnp.zeros((xl.shape[0], 13), _f32)
    return jnp.concatenate([xr] + ls + [pad], axis=1)


def _loop_attr(laa, lab):
    la = laa + lab
    cnt = jnp.maximum(la[:, 4:5], 1.0)
    return la[:, 0:4] / cnt


def _tc_b1_body(x_r, ntm_r, laa_r, lab_r, W4_r, b4_r, Wl_r, bl_r, Wr_r, br_r,
                We_r, att_r, xl_o, xr_o):
    xb = x_r[...]
    ntm = ntm_r[...]
    h = jnp.zeros((R, HID), _f32)
    for t in range(4):
        proj = jnp.dot(xb, W4_r[t], preferred_element_type=_f32) + b4_r[t]
        h = jnp.where(ntm == t, proj, h)
    h = jnp.maximum(h, 0.0)
    xl = jnp.dot(h, Wl_r[0], preferred_element_type=_f32) + bl_r[0, 0]
    xr = jnp.dot(h, Wr_r[0], preferred_element_type=_f32) + br_r[0, 0]
    el = jnp.dot(_loop_attr(laa_r[...], lab_r[...]), We_r[0],
                 preferred_element_type=_f32)
    xl_o[...] = xl
    xr_o[...] = _lself_cols(xl, xr, el, att_r[0, 0])


def _tc_b1(x, ntm3, la, W4, b4, Wl, bl2, Wr, br2, We, att2):
    half = lambda c, i: (c * NB + i, 0)
    row = lambda c, i: (i, 0)
    return pl.pallas_call(
        _tc_b1_body,
        grid=(2, NB),
        in_specs=[
            pl.BlockSpec((R, IN_DIM), row),
            pl.BlockSpec((R, 1), lambda c, i: (i, 0)),
            pl.BlockSpec((R, 16), lambda c, i: (i, 0)),
            pl.BlockSpec((R, 16), lambda c, i: (NB + i, 0)),
            pl.BlockSpec((4, IN_DIM, HID), lambda c, i: (0, 0, 0)),
            pl.BlockSpec((4, HID), lambda c, i: (0, 0)),
            pl.BlockSpec((1, HID, H2), lambda c, i: (c, 0, 0)),
            pl.BlockSpec((1, 1, H2), lambda c, i: (c, 0, 0)),
            pl.BlockSpec((1, HID, H2), lambda c, i: (c, 0, 0)),
            pl.BlockSpec((1, 1, H2), lambda c, i: (c, 0, 0)),
            pl.BlockSpec((1, 4, H2), lambda c, i: (c, 0, 0)),
            pl.BlockSpec((1, 1, H2), lambda c, i: (c, 0, 0)),
        ],
        out_specs=[pl.BlockSpec((R, H2), half), pl.BlockSpec((R, AW), half)],
        out_shape=[jax.ShapeDtypeStruct((2 * N, H2), _f32),
                   jax.ShapeDtypeStruct((2 * N, AW), _f32)],
    )(x, ntm3, la, la, W4, b4, Wl, bl2, Wr, br2, We, att2)


def _combine_h(acca, accb, dena, denb, xla, xlb, bias2):
    chunks = []
    for c in range(2):
        acc = acca if c == 0 else accb
        den = dena if c == 0 else denb
        xlh = xla if c == 0 else xlb
        num = acc + xlh
        for u in range(3):
            d = den[:, u:u + 1] + 1.0
            hc = num[:, u * HID:(u + 1) * HID] / d + bias2[c, u * HID:(u + 1) * HID]
            chunks.append(jnp.maximum(hc, 0.0))
    return jnp.concatenate(chunks, axis=1)


def _tc_b2_body(acca_r, accb_r, dena_r, denb_r, xla_r, xlb_r, bias_r,
                laa_r, lab_r,
                Wl_r, bl_r, Wr_r, br_r, We_r, att_r, xl_o, xr_o):
    h = _combine_h(acca_r[...], accb_r[...], dena_r[...], denb_r[...],
                   xla_r[...], xlb_r[...], bias_r[:, 0, :])
    xl = jnp.dot(h, Wl_r[0], preferred_element_type=_f32) + bl_r[0, 0]
    xr = jnp.dot(h, Wr_r[0], preferred_element_type=_f32) + br_r[0, 0]
    el = jnp.dot(_loop_attr(laa_r[...], lab_r[...]), We_r[0],
                 preferred_element_type=_f32)
    xl_o[...] = xl
    xr_o[...] = _lself_cols(xl, xr, el, att_r[0, 0])


def _tc_b2(acc1, den1, xl1, bias1_2, la, Wl, bl2, Wr, br2, We, att2):
    half = lambda c, i: (c * NB + i, 0)
    return pl.pallas_call(
        _tc_b2_body,
        grid=(2, NB),
        in_specs=[
            pl.BlockSpec((R, H2), lambda c, i: (i, 0)),
            pl.BlockSpec((R, H2), lambda c, i: (NB + i, 0)),
            pl.BlockSpec((R, 16), lambda c, i: (i, 0)),
            pl.BlockSpec((R, 16), lambda c, i: (NB + i, 0)),
            pl.BlockSpec((R, H2), lambda c, i: (i, 0)),
            pl.BlockSpec((R, H2), lambda c, i: (NB + i, 0)),
            pl.BlockSpec((2, 1, H2), lambda c, i: (0, 0, 0)),
            pl.BlockSpec((R, 16), lambda c, i: (i, 0)),
            pl.BlockSpec((R, 16), lambda c, i: (NB + i, 0)),
            pl.BlockSpec((1, HC, H2), lambda c, i: (c, 0, 0)),
            pl.BlockSpec((1, 1, H2), lambda c, i: (c, 0, 0)),
            pl.BlockSpec((1, HC, H2), lambda c, i: (c, 0, 0)),
            pl.BlockSpec((1, 1, H2), lambda c, i: (c, 0, 0)),
            pl.BlockSpec((1, 4, H2), lambda c, i: (c, 0, 0)),
            pl.BlockSpec((1, 1, H2), lambda c, i: (c, 0, 0)),
        ],
        out_specs=[pl.BlockSpec((R, H2), half), pl.BlockSpec((R, AW), half)],
        out_shape=[jax.ShapeDtypeStruct((2 * N, H2), _f32),
                   jax.ShapeDtypeStruct((2 * N, AW), _f32)],
    )(acc1, acc1, den1, den1, xl1, xl1, bias1_2, la, la,
      Wl, bl2, Wr, br2, We, att2)


def _tc_eemb_body(ea_r, We_r, out_o):
    out_o[...] = jnp.dot(ea_r[...], We_r[0], preferred_element_type=_f32)


def _tc_eemb(ea, We):
    return pl.pallas_call(
        _tc_eemb_body,
        grid=(2, NEB),
        in_specs=[
            pl.BlockSpec((EB, 4), lambda c, i: (i, 0)),
            pl.BlockSpec((1, 4, H2), lambda c, i: (c, 0, 0)),
        ],
        out_specs=pl.BlockSpec((EB, H2), lambda c, i: (c * NEB + i, 0)),
        out_shape=jax.ShapeDtypeStruct((2 * E, H2), _f32),
    )(ea, We)


def _tc_pool_body(acca_r, accb_r, dena_r, denb_r, xla_r, xlb_r, bias_r,
                  bat_r, Wo_r, bo_r, out_o, sums, cnts):
    i = pl.program_id(0)

    @pl.when(i == 0)
    def _():
        sums[...] = jnp.zeros((NUM_GRAPHS, HC), _f32)
        cnts[...] = jnp.zeros((NUM_GRAPHS, 128), _f32)

    h = _combine_h(acca_r[...], accb_r[...], dena_r[...], denb_r[...],
                   xla_r[...], xlb_r[...], bias_r[:, 0, :])
    bat = bat_r[0]
    gid = lax.broadcasted_iota(jnp.int32, (NUM_GRAPHS, R), 0)
    onehot = (bat == gid).astype(_f32)
    sums[...] += jnp.dot(onehot, h, preferred_element_type=_f32)
    cnts[...] += jnp.dot(onehot, jnp.ones((R, 128), _f32),
                         preferred_element_type=_f32)

    @pl.when(i == NB - 1)
    def _():
        pooled = sums[...] / jnp.maximum(cnts[:, 0:1], 1.0)
        out_o[...] = jnp.tanh(
            jnp.dot(pooled, Wo_r[...], preferred_element_type=_f32) + bo_r[0])


def _tc_pool(acc2, den2, xl2, bias2_2, bat3, Wo, bo2):
    return pl.pallas_call(
        _tc_pool_body,
        grid=(NB,),
        in_specs=[
            pl.BlockSpec((R, H2), lambda i: (i, 0)),
            pl.BlockSpec((R, H2), lambda i: (NB + i, 0)),
            pl.BlockSpec((R, 16), lambda i: (i, 0)),
            pl.BlockSpec((R, 16), lambda i: (NB + i, 0)),
            pl.BlockSpec((R, H2), lambda i: (i, 0)),
            pl.BlockSpec((R, H2), lambda i: (NB + i, 0)),
            pl.BlockSpec((2, 1, H2), lambda i: (0, 0, 0)),
            pl.BlockSpec((1, 1, R), lambda i: (i, 0, 0)),
            pl.BlockSpec((HC, OUT_DIM), lambda i: (0, 0)),
            pl.BlockSpec((1, OUT_DIM), lambda i: (0, 0)),
        ],
        out_specs=pl.BlockSpec((NUM_GRAPHS, OUT_DIM), lambda i: (0, 0)),
        out_shape=jax.ShapeDtypeStruct((NUM_GRAPHS, OUT_DIM), _f32),
        scratch_shapes=[pltpu.VMEM((NUM_GRAPHS, HC), _f32),
                        pltpu.VMEM((NUM_GRAPHS, 128), _f32)],
    )(acc2, acc2, den2, den2, xl2, xl2, bias2_2, bat3, Wo, bo2)


def kernel(x, edge_index, edge_attr, node_type_mask, batch, params):
    src = edge_index[0].astype(jnp.int32)
    dst = edge_index[1].astype(jnp.int32)
    ea_pad = jnp.concatenate(
        [edge_attr, jnp.ones((E, 1), _f32), jnp.zeros((E, 11), _f32)], axis=1)
    ntm3 = node_type_mask.astype(jnp.int32).reshape(N, 1)
    bat3 = batch.astype(jnp.int32).reshape(NB, 1, R)
    z16 = jnp.zeros((ZR, 16), _f32)
    z192 = jnp.zeros((BE, H2), _f32)

    p = params
    W4 = jnp.stack([p["W_joint"], p["W_obj"], p["W_tcp"], p["W_goal"]])
    b4 = jnp.stack([p["b_joint"], p["b_obj"], p["b_tcp"], p["b_goal"]])
    c1, c2 = p["convs"][0], p["convs"][1]

    la = _sc_loopattr(ea_pad, dst, z16)

    att1 = c1["att"].reshape(2, 1, H2)
    xl1, xr1 = _tc_b1(x, ntm3, la, W4, b4,
                      _halves(c1["Wl"]), c1["bl"].reshape(2, 1, H2),
                      _halves(c1["Wr"]), c1["br"].reshape(2, 1, H2),
                      _halves(c1["We"]), att1)
    em1 = _tc_eemb(edge_attr, _halves(c1["We"]))
    acc1, w1 = _sc_edge(xl1, xr1, em1, src, dst, att1, z192)
    den1 = _sc_den(w1, dst, z16)

    att2 = c2["att"].reshape(2, 1, H2)
    xl2, xr2 = _tc_b2(acc1, den1, xl1, c1["bias"].reshape(2, 1, H2), la,
                      _halves(c2["Wl"]), c2["bl"].reshape(2, 1, H2),
                      _halves(c2["Wr"]), c2["br"].reshape(2, 1, H2),
                      _halves(c2["We"]), att2)
    em2 = _tc_eemb(edge_attr, _halves(c2["We"]))
    acc2, w2 = _sc_edge(xl2, xr2, em2, src, dst, att2, z192)
    den2 = _sc_den(w2, dst, z16)

    return _tc_pool(acc2, den2, xl2, c2["bias"].reshape(2, 1, H2), bat3,
                    p["W_out"], p["b_out"].reshape(1, OUT_DIM))


# speedup vs baseline: 7.9520x; 1.3385x over previous
import functools

import jax
import jax.numpy as jnp
from jax import lax
from jax.experimental import pallas as pl
from jax.experimental.pallas import tpu as pltpu
from jax.experimental.pallas import tpu_sc as plsc

N = 10000
E = 160000
IN_DIM = 128
HID = 64
HEADS = 6
HC = HID * HEADS
OUT_DIM = 32
NUM_GRAPHS = 64

H2 = HC // 2
AW = H2 + 16
R = 1000
NB = N // R
EB = 2000
NEB = E // EB
TPS = N // 16
ZR = 125
B = 80
BE = 16
EPT = E // 16
NBAT = EPT // B
B0 = 40

_f32 = jnp.float32


def _halves(W):
    return W.reshape(W.shape[0], 2, H2).transpose(1, 0, 2)


def _sc_loopattr_body(ea_hbm, dst_hbm, z16_hbm, out_hbm,
                      la_sh, ea_v, dst_v, zer_v, sem_i, sem_sc):
    c = lax.axis_index("c")
    s = lax.axis_index("s")
    pltpu.sync_copy(z16_hbm, zer_v)
    for r in range(TPS // ZR):
        pltpu.sync_copy(zer_v, la_sh.at[pl.ds(s * TPS + r * ZR, ZR)])
    plsc.subcore_barrier()
    wid = s * 2 + c
    ebase = wid * (E // 32)

    def batch(b, carry):
        @pl.when(b > 0)
        def _():
            pltpu.make_async_copy(ea_v, la_sh.at[dst_v], sem_sc).wait()
        base = ebase + b * B0
        d1 = pltpu.async_copy(dst_hbm.at[pl.ds(base, B0)], dst_v, sem_i)
        d2 = pltpu.async_copy(ea_hbm.at[pl.ds(base, B0)], ea_v, sem_i)
        d1.wait()
        d2.wait()
        pltpu.async_copy(ea_v, la_sh.at[dst_v], sem_sc, add=True)
        return carry

    lax.fori_loop(0, (E // 32) // B0, batch, 0)
    pltpu.make_async_copy(ea_v, la_sh.at[dst_v], sem_sc).wait()
    plsc.subcore_barrier()
    for r in range(TPS // ZR):
        rows = s * TPS + r * ZR
        pltpu.sync_copy(la_sh.at[pl.ds(rows, ZR)], zer_v)
        pltpu.sync_copy(zer_v, out_hbm.at[pl.ds(c * N + rows, ZR)])


def _sc_loopattr(ea_pad, dst, z16):
    mesh = plsc.VectorSubcoreMesh(core_axis_name="c", subcore_axis_name="s")
    return pl.kernel(
        _sc_loopattr_body,
        out_type=jax.ShapeDtypeStruct((2 * N, 16), _f32),
        mesh=mesh,
        compiler_params=pltpu.CompilerParams(needs_layout_passes=False, use_tc_tiling_on_sc=False),
        scratch_types=[
            pltpu.VMEM_SHARED((N, 16), _f32),
            pltpu.VMEM((B0, 16), _f32),
            pltpu.VMEM((B0,), jnp.int32),
            pltpu.VMEM((ZR, 16), _f32),
            pltpu.SemaphoreType.DMA,
            pltpu.SemaphoreType.DMA,
        ],
    )(ea_pad, dst, z16)


def _sc_edge_body(xl_hbm, xr_hbm, em_hbm, src_hbm, dst_hbm, att_hbm, z192_hbm,
                  out_hbm, w_hbm,
                  acc_sh, src_v, dst_v, srcg_v, dstg_v,
                  xl_v, xr_v, em_v, w_v, att_v, red_v,
                  sem_i, sem_g, sem_sc, sem_w):
    c = lax.axis_index("c")
    s = lax.axis_index("s")
    cN = c * N
    pltpu.sync_copy(z192_hbm, xl_v)
    for r in range(TPS // BE):
        pltpu.sync_copy(xl_v, acc_sh.at[pl.ds(s * TPS + r * BE, BE)])
    pltpu.sync_copy(xl_v.at[pl.ds(0, TPS % BE)],
                    acc_sh.at[pl.ds(s * TPS + (TPS // BE) * BE, TPS % BE)])
    pltpu.sync_copy(att_hbm.at[c, 0], att_v)
    plsc.subcore_barrier()
    ebase = s * EPT
    iota = lax.iota(jnp.int32, 16)

    def batch(b, carry):
        base = ebase + b * BE

        @pl.when(b > 0)
        def _():
            pltpu.make_async_copy(xl_v, acc_sh.at[dst_v], sem_sc).wait()
            pltpu.make_async_copy(
                w_v, w_hbm.at[pl.ds(c * E + base - BE, BE)], sem_w).wait()

        d1 = pltpu.async_copy(src_hbm.at[pl.ds(base, BE)], src_v, sem_i)
        d2 = pltpu.async_copy(dst_hbm.at[pl.ds(base, BE)], dst_v, sem_i)
        d1.wait()
        d2.wait()
        srcg_v[pl.ds(0, 16)] = src_v[pl.ds(0, 16)] + cN
        dstg_v[pl.ds(0, 16)] = dst_v[pl.ds(0, 16)] + cN
        g1 = pltpu.async_copy(xl_hbm.at[srcg_v], xl_v, sem_g)
        g2 = pltpu.async_copy(xr_hbm.at[dstg_v], xr_v, sem_g)
        g3 = pltpu.async_copy(em_hbm.at[pl.ds(c * E + base, BE)], em_v, sem_g)
        g1.wait()
        g2.wait()
        g3.wait()

        def edge(i, carry2):
            irow = iota * 0 + i
            ws = []
            for u in range(3):
                acc = None
                for q in range(4):
                    j = u * 4 + q
                    t = (xl_v[i, pl.ds(j * 16, 16)]
                         + xr_v[i, pl.ds(j * 16, 16)]
                         + em_v[i, pl.ds(j * 16, 16)])
                    t = jnp.maximum(t, 0.2 * t)
                    t = t * att_v[pl.ds(j * 16, 16)]
                    acc = t if acc is None else acc + t
                for k in (8, 4, 2, 1):
                    red_v[pl.ds(u * 16, 16)] = acc
                    acc = acc + plsc.load_gather(
                        red_v, [u * 16 + jnp.bitwise_xor(iota, k)])
                lself_u = plsc.load_gather(
                    xr_v, [irow, jnp.full((16,), H2 + u, jnp.int32)])
                ws.append(jnp.exp(acc - lself_u))
            w_v[i, pl.ds(0, 16)] = jnp.where(
                iota == 0, ws[0], jnp.where(iota == 1, ws[1],
                                            jnp.where(iota == 2, ws[2], 0.0)))
            for u in range(3):
                for q in range(4):
                    j = u * 4 + q
                    xl_v[i, pl.ds(j * 16, 16)] = (
                        xl_v[i, pl.ds(j * 16, 16)] * ws[u])
            return carry2

        lax.fori_loop(0, BE, edge, 0)
        pltpu.async_copy(xl_v, acc_sh.at[dst_v], sem_sc, add=True)
        pltpu.async_copy(w_v, w_hbm.at[pl.ds(c * E + base, BE)], sem_w)
        return carry

    lax.fori_loop(0, EPT // BE, batch, 0)
    pltpu.make_async_copy(xl_v, acc_sh.at[dst_v], sem_sc).wait()
    pltpu.make_async_copy(
        w_v, w_hbm.at[pl.ds(c * E + EPT - BE, BE)], sem_w).wait()
    plsc.subcore_barrier()
    for r in range(TPS // BE):
        rows = s * TPS + r * BE
        pltpu.sync_copy(acc_sh.at[pl.ds(rows, BE)], xl_v)
        pltpu.sync_copy(xl_v, out_hbm.at[pl.ds(cN + rows, BE)])
    rows = s * TPS + (TPS // BE) * BE
    pltpu.sync_copy(acc_sh.at[pl.ds(rows, TPS % BE)], xl_v.at[pl.ds(0, TPS % BE)])
    pltpu.sync_copy(xl_v.at[pl.ds(0, TPS % BE)], out_hbm.at[pl.ds(cN + rows, TPS % BE)])


def _sc_edge(xl, xr_aug, eemb, src, dst, att2, z192):
    mesh = plsc.VectorSubcoreMesh(core_axis_name="c", subcore_axis_name="s")
    return pl.kernel(
        _sc_edge_body,
        out_type=[jax.ShapeDtypeStruct((2 * N, H2), _f32),
                  jax.ShapeDtypeStruct((2 * E, 16), _f32)],
        mesh=mesh,
        compiler_params=pltpu.CompilerParams(needs_layout_passes=False, use_tc_tiling_on_sc=False),
        scratch_types=[
            pltpu.VMEM_SHARED((N, H2), _f32),
            pltpu.VMEM((BE,), jnp.int32),
            pltpu.VMEM((BE,), jnp.int32),
            pltpu.VMEM((BE,), jnp.int32),
            pltpu.VMEM((BE,), jnp.int32),
            pltpu.VMEM((BE, H2), _f32),
            pltpu.VMEM((BE, AW), _f32),
            pltpu.VMEM((BE, H2), _f32),
            pltpu.VMEM((BE, 16), _f32),
            pltpu.VMEM((H2,), _f32),
            pltpu.VMEM((48,), _f32),
            pltpu.SemaphoreType.DMA,
            pltpu.SemaphoreType.DMA,
            pltpu.SemaphoreType.DMA,
            pltpu.SemaphoreType.DMA,
        ],
    )(xl, xr_aug, eemb, src, dst, att2, z192)


def _sc_den_body(w_hbm, dst_hbm, z16_hbm, out_hbm,
                 den_sh, w_v, dst_v, zer_v, sem_i, sem_sc):
    c = lax.axis_index("c")
    s = lax.axis_index("s")
    pltpu.sync_copy(z16_hbm, zer_v)
    for r in range(TPS // ZR):
        pltpu.sync_copy(zer_v, den_sh.at[pl.ds(s * TPS + r * ZR, ZR)])
    plsc.subcore_barrier()
    ebase = s * EPT

    def batch(b, carry):
        @pl.when(b > 0)
        def _():
            pltpu.make_async_copy(w_v, den_sh.at[dst_v], sem_sc).wait()
        base = ebase + b * B
        d1 = pltpu.async_copy(dst_hbm.at[pl.ds(base, B)], dst_v, sem_i)
        d2 = pltpu.async_copy(w_hbm.at[pl.ds(c * E + base, B)], w_v, sem_i)
        d1.wait()
        d2.wait()
        pltpu.async_copy(w_v, den_sh.at[dst_v], sem_sc, add=True)
        return carry

    lax.fori_loop(0, EPT // B, batch, 0)
    pltpu.make_async_copy(w_v, den_sh.at[dst_v], sem_sc).wait()
    plsc.subcore_barrier()
    for r in range(TPS // ZR):
        rows = s * TPS + r * ZR
        pltpu.sync_copy(den_sh.at[pl.ds(rows, ZR)], zer_v)
        pltpu.sync_copy(zer_v, out_hbm.at[pl.ds(c * N + rows, ZR)])


def _sc_den(w, dst, z16):
    mesh = plsc.VectorSubcoreMesh(core_axis_name="c", subcore_axis_name="s")
    return pl.kernel(
        _sc_den_body,
        out_type=jax.ShapeDtypeStruct((2 * N, 16), _f32),
        mesh=mesh,
        compiler_params=pltpu.CompilerParams(needs_layout_passes=False, use_tc_tiling_on_sc=False),
        scratch_types=[
            pltpu.VMEM_SHARED((N, 16), _f32),
            pltpu.VMEM((B, 16), _f32),
            pltpu.VMEM((B,), jnp.int32),
            pltpu.VMEM((ZR, 16), _f32),
            pltpu.SemaphoreType.DMA,
            pltpu.SemaphoreType.DMA,
        ],
    )(w, dst, z16)


def _lself_cols(xl, xr, el, att_row):
    t = xl + xr + el
    t = jnp.maximum(t, 0.2 * t)
    ta = t * att_row[None, :]
    ls = [jnp.sum(ta[:, u * HID:(u + 1) * HID], axis=1, keepdims=True)
          for u in range(3)]
    pad = jnp.zeros((xl.shape[0], 13), _f32)
    return jnp.concatenate([xr] + ls + [pad], axis=1)


def _loop_attr(laa, lab):
    la = laa + lab
    cnt = jnp.maximum(la[:, 4:5], 1.0)
    return la[:, 0:4] / cnt


def _tc_b1_body(x_r, ntm_r, laa_r, lab_r, W4_r, b4_r, Wl_r, bl_r, Wr_r, br_r,
                We_r, att_r, xl_o, xr_o):
    xb = x_r[...]
    ntm = ntm_r[...]
    h = jnp.zeros((R, HID), _f32)
    for t in range(4):
        proj = jnp.dot(xb, W4_r[t], preferred_element_type=_f32) + b4_r[t]
        h = jnp.where(ntm == t, proj, h)
    h = jnp.maximum(h, 0.0)
    xl = jnp.dot(h, Wl_r[0], preferred_element_type=_f32) + bl_r[0, 0]
    xr = jnp.dot(h, Wr_r[0], preferred_element_type=_f32) + br_r[0, 0]
    el = jnp.dot(_loop_attr(laa_r[...], lab_r[...]), We_r[0],
                 preferred_element_type=_f32)
    xl_o[...] = xl
    xr_o[...] = _lself_cols(xl, xr, el, att_r[0, 0])


def _tc_b1(x, ntm3, la, W4, b4, Wl, bl2, Wr, br2, We, att2):
    half = lambda c, i: (c * NB + i, 0)
    row = lambda c, i: (i, 0)
    return pl.pallas_call(
        _tc_b1_body,
        grid=(2, NB),
        in_specs=[
            pl.BlockSpec((R, IN_DIM), row),
            pl.BlockSpec((R, 1), lambda c, i: (i, 0)),
            pl.BlockSpec((R, 16), lambda c, i: (i, 0)),
            pl.BlockSpec((R, 16), lambda c, i: (NB + i, 0)),
            pl.BlockSpec((4, IN_DIM, HID), lambda c, i: (0, 0, 0)),
            pl.BlockSpec((4, HID), lambda c, i: (0, 0)),
            pl.BlockSpec((1, HID, H2), lambda c, i: (c, 0, 0)),
            pl.BlockSpec((1, 1, H2), lambda c, i: (c, 0, 0)),
            pl.BlockSpec((1, HID, H2), lambda c, i: (c, 0, 0)),
            pl.BlockSpec((1, 1, H2), lambda c, i: (c, 0, 0)),
            pl.BlockSpec((1, 4, H2), lambda c, i: (c, 0, 0)),
            pl.BlockSpec((1, 1, H2), lambda c, i: (c, 0, 0)),
        ],
        out_specs=[pl.BlockSpec((R, H2), half), pl.BlockSpec((R, AW), half)],
        out_shape=[jax.ShapeDtypeStruct((2 * N, H2), _f32),
                   jax.ShapeDtypeStruct((2 * N, AW), _f32)],
    )(x, ntm3, la, la, W4, b4, Wl, bl2, Wr, br2, We, att2)


def _combine_h(acca, accb, dena, denb, xla, xlb, bias2):
    chunks = []
    for c in range(2):
        acc = acca if c == 0 else accb
        den = dena if c == 0 else denb
        xlh = xla if c == 0 else xlb
        num = acc + xlh
        for u in range(3):
            d = den[:, u:u + 1] + 1.0
            hc = num[:, u * HID:(u + 1) * HID] / d + bias2[c, u * HID:(u + 1) * HID]
            chunks.append(jnp.maximum(hc, 0.0))
    return jnp.concatenate(chunks, axis=1)


def _tc_b2_body(acca_r, accb_r, dena_r, denb_r, xla_r, xlb_r, bias_r,
                laa_r, lab_r,
                Wl_r, bl_r, Wr_r, br_r, We_r, att_r, xl_o, xr_o):
    h = _combine_h(acca_r[...], accb_r[...], dena_r[...], denb_r[...],
                   xla_r[...], xlb_r[...], bias_r[:, 0, :])
    xl = jnp.dot(h, Wl_r[0], preferred_element_type=_f32) + bl_r[0, 0]
    xr = jnp.dot(h, Wr_r[0], preferred_element_type=_f32) + br_r[0, 0]
    el = jnp.dot(_loop_attr(laa_r[...], lab_r[...]), We_r[0],
                 preferred_element_type=_f32)
    xl_o[...] = xl
    xr_o[...] = _lself_cols(xl, xr, el, att_r[0, 0])


def _tc_b2(acc1, den1, xl1, bias1_2, la, Wl, bl2, Wr, br2, We, att2):
    half = lambda c, i: (c * NB + i, 0)
    return pl.pallas_call(
        _tc_b2_body,
        grid=(2, NB),
        in_specs=[
            pl.BlockSpec((R, H2), lambda c, i: (i, 0)),
            pl.BlockSpec((R, H2), lambda c, i: (NB + i, 0)),
            pl.BlockSpec((R, 16), lambda c, i: (i, 0)),
            pl.BlockSpec((R, 16), lambda c, i: (NB + i, 0)),
            pl.BlockSpec((R, H2), lambda c, i: (i, 0)),
            pl.BlockSpec((R, H2), lambda c, i: (NB + i, 0)),
            pl.BlockSpec((2, 1, H2), lambda c, i: (0, 0, 0)),
            pl.BlockSpec((R, 16), lambda c, i: (i, 0)),
            pl.BlockSpec((R, 16), lambda c, i: (NB + i, 0)),
            pl.BlockSpec((1, HC, H2), lambda c, i: (c, 0, 0)),
            pl.BlockSpec((1, 1, H2), lambda c, i: (c, 0, 0)),
            pl.BlockSpec((1, HC, H2), lambda c, i: (c, 0, 0)),
            pl.BlockSpec((1, 1, H2), lambda c, i: (c, 0, 0)),
            pl.BlockSpec((1, 4, H2), lambda c, i: (c, 0, 0)),
            pl.BlockSpec((1, 1, H2), lambda c, i: (c, 0, 0)),
        ],
        out_specs=[pl.BlockSpec((R, H2), half), pl.BlockSpec((R, AW), half)],
        out_shape=[jax.ShapeDtypeStruct((2 * N, H2), _f32),
                   jax.ShapeDtypeStruct((2 * N, AW), _f32)],
    )(acc1, acc1, den1, den1, xl1, xl1, bias1_2, la, la,
      Wl, bl2, Wr, br2, We, att2)


def _tc_eemb_body(ea_r, We_r, out_o):
    out_o[...] = jnp.dot(ea_r[...], We_r[0], preferred_element_type=_f32)


def _tc_eemb(ea, We):
    return pl.pallas_call(
        _tc_eemb_body,
        grid=(2, NEB),
        in_specs=[
            pl.BlockSpec((EB, 4), lambda c, i: (i, 0)),
            pl.BlockSpec((1, 4, H2), lambda c, i: (c, 0, 0)),
        ],
        out_specs=pl.BlockSpec((EB, H2), lambda c, i: (c * NEB + i, 0)),
        out_shape=jax.ShapeDtypeStruct((2 * E, H2), _f32),
    )(ea, We)


def _tc_pool_body(acca_r, accb_r, dena_r, denb_r, xla_r, xlb_r, bias_r,
                  bat_r, Wo_r, bo_r, out_o, sums, cnts):
    i = pl.program_id(0)

    @pl.when(i == 0)
    def _():
        sums[...] = jnp.zeros((NUM_GRAPHS, HC), _f32)
        cnts[...] = jnp.zeros((NUM_GRAPHS, 128), _f32)

    h = _combine_h(acca_r[...], accb_r[...], dena_r[...], denb_r[...],
                   xla_r[...], xlb_r[...], bias_r[:, 0, :])
    bat = bat_r[0]
    gid = lax.broadcasted_iota(jnp.int32, (NUM_GRAPHS, R), 0)
    onehot = (bat == gid).astype(_f32)
    sums[...] += jnp.dot(onehot, h, preferred_element_type=_f32)
    cnts[...] += jnp.dot(onehot, jnp.ones((R, 128), _f32),
                         preferred_element_type=_f32)

    @pl.when(i == NB - 1)
    def _():
        pooled = sums[...] / jnp.maximum(cnts[:, 0:1], 1.0)
        out_o[...] = jnp.tanh(
            jnp.dot(pooled, Wo_r[...], preferred_element_type=_f32) + bo_r[0])


def _tc_pool(acc2, den2, xl2, bias2_2, bat3, Wo, bo2):
    return pl.pallas_call(
        _tc_pool_body,
        grid=(NB,),
        in_specs=[
            pl.BlockSpec((R, H2), lambda i: (i, 0)),
            pl.BlockSpec((R, H2), lambda i: (NB + i, 0)),
            pl.BlockSpec((R, 16), lambda i: (i, 0)),
            pl.BlockSpec((R, 16), lambda i: (NB + i, 0)),
            pl.BlockSpec((R, H2), lambda i: (i, 0)),
            pl.BlockSpec((R, H2), lambda i: (NB + i, 0)),
            pl.BlockSpec((2, 1, H2), lambda i: (0, 0, 0)),
            pl.BlockSpec((1, 1, R), lambda i: (i, 0, 0)),
            pl.BlockSpec((HC, OUT_DIM), lambda i: (0, 0)),
            pl.BlockSpec((1, OUT_DIM), lambda i: (0, 0)),
        ],
        out_specs=pl.BlockSpec((NUM_GRAPHS, OUT_DIM), lambda i: (0, 0)),
        out_shape=jax.ShapeDtypeStruct((NUM_GRAPHS, OUT_DIM), _f32),
        scratch_shapes=[pltpu.VMEM((NUM_GRAPHS, HC), _f32),
                        pltpu.VMEM((NUM_GRAPHS, 128), _f32)],
    )(acc2, acc2, den2, den2, xl2, xl2, bias2_2, bat3, Wo, bo2)


def kernel(x, edge_index, edge_attr, node_type_mask, batch, params):
    src = edge_index[0].astype(jnp.int32)
    dst = edge_index[1].astype(jnp.int32)
    ea_pad = jnp.concatenate(
        [edge_attr, jnp.ones((E, 1), _f32), jnp.zeros((E, 11), _f32)], axis=1)
    ntm3 = node_type_mask.astype(jnp.int32).reshape(N, 1)
    bat3 = batch.astype(jnp.int32).reshape(NB, 1, R)
    z16 = jnp.zeros((ZR, 16), _f32)
    z192 = jnp.zeros((BE, H2), _f32)

    p = params
    W4 = jnp.stack([p["W_joint"], p["W_obj"], p["W_tcp"], p["W_goal"]])
    b4 = jnp.stack([p["b_joint"], p["b_obj"], p["b_tcp"], p["b_goal"]])
    c1, c2 = p["convs"][0], p["convs"][1]

    la = _sc_loopattr(ea_pad, dst, z16)

    att1 = c1["att"].reshape(2, 1, H2)
    xl1, xr1 = _tc_b1(x, ntm3, la, W4, b4,
                      _halves(c1["Wl"]), c1["bl"].reshape(2, 1, H2),
                      _halves(c1["Wr"]), c1["br"].reshape(2, 1, H2),
                      _halves(c1["We"]), att1)
    em1 = _tc_eemb(edge_attr, _halves(c1["We"]))
    acc1, w1 = _sc_edge(xl1, xr1, em1, src, dst, att1, z192)
    den1 = _sc_den(w1, dst, z16)

    att2 = c2["att"].reshape(2, 1, H2)
    xl2, xr2 = _tc_b2(acc1, den1, xl1, c1["bias"].reshape(2, 1, H2), la,
                      _halves(c2["Wl"]), c2["bl"].reshape(2, 1, H2),
                      _halves(c2["Wr"]), c2["br"].reshape(2, 1, H2),
                      _halves(c2["We"]), att2)
    em2 = _tc_eemb(edge_attr, _halves(c2["We"]))
    acc2, w2 = _sc_edge(xl2, xr2, em2, src, dst, att2, z192)
    den2 = _sc_den(w2, dst, z16)

    return _tc_pool(acc2, den2, xl2, c2["bias"].reshape(2, 1, H2), bat3,
                    p["W_out"], p["b_out"].reshape(1, OUT_DIM))


# speedup vs baseline: 10.9999x; 1.3833x over previous
import functools

import jax
import jax.numpy as jnp
from jax import lax
from jax.experimental import pallas as pl
from jax.experimental.pallas import tpu as pltpu
from jax.experimental.pallas import tpu_sc as plsc

N = 10000
E = 160000
IN_DIM = 128
HID = 64
HEADS = 6
HC = HID * HEADS
OUT_DIM = 32
NUM_GRAPHS = 64

H2 = HC // 2
AW = H2 + 16
R = 1000
NB = N // R
EB = 2000
NEB = E // EB
TPS = N // 16
ZR = 125
B = 80
BE = 16
EPT = E // 16
NBAT = EPT // B
B0 = 40

_f32 = jnp.float32


def _halves(W):
    return W.reshape(W.shape[0], 2, H2).transpose(1, 0, 2)


def _sc_loopattr_body(ea_hbm, dst_hbm, z16_hbm, out_hbm,
                      la_sh, ea_v, dst_v, zer_v, sem_i, sem_sc):
    c = lax.axis_index("c")
    s = lax.axis_index("s")
    pltpu.sync_copy(z16_hbm, zer_v)
    for r in range(TPS // ZR):
        pltpu.sync_copy(zer_v, la_sh.at[pl.ds(s * TPS + r * ZR, ZR)])
    plsc.subcore_barrier()
    wid = s * 2 + c
    ebase = wid * (E // 32)

    def batch(b, carry):
        @pl.when(b > 0)
        def _():
            pltpu.make_async_copy(ea_v, la_sh.at[dst_v], sem_sc).wait()
        base = ebase + b * B0
        d1 = pltpu.async_copy(dst_hbm.at[pl.ds(base, B0)], dst_v, sem_i)
        d2 = pltpu.async_copy(ea_hbm.at[pl.ds(base, B0)], ea_v, sem_i)
        d1.wait()
        d2.wait()
        pltpu.async_copy(ea_v, la_sh.at[dst_v], sem_sc, add=True)
        return carry

    lax.fori_loop(0, (E // 32) // B0, batch, 0)
    pltpu.make_async_copy(ea_v, la_sh.at[dst_v], sem_sc).wait()
    plsc.subcore_barrier()
    for r in range(TPS // ZR):
        rows = s * TPS + r * ZR
        pltpu.sync_copy(la_sh.at[pl.ds(rows, ZR)], zer_v)
        pltpu.sync_copy(zer_v, out_hbm.at[pl.ds(c * N + rows, ZR)])


def _sc_loopattr(ea_pad, dst, z16):
    mesh = plsc.VectorSubcoreMesh(core_axis_name="c", subcore_axis_name="s")
    return pl.kernel(
        _sc_loopattr_body,
        out_type=jax.ShapeDtypeStruct((2 * N, 16), _f32),
        mesh=mesh,
        compiler_params=pltpu.CompilerParams(needs_layout_passes=False, use_tc_tiling_on_sc=False),
        scratch_types=[
            pltpu.VMEM_SHARED((N, 16), _f32),
            pltpu.VMEM((B0, 16), _f32),
            pltpu.VMEM((B0,), jnp.int32),
            pltpu.VMEM((ZR, 16), _f32),
            pltpu.SemaphoreType.DMA,
            pltpu.SemaphoreType.DMA,
        ],
    )(ea_pad, dst, z16)


SB = 8
NSB = EPT // SB
NP = NSB // 2


def _sc_edge_body(xl_hbm, xr_hbm, em_hbm, src_hbm, dst_hbm, att_hbm, z192_hbm,
                  out_hbm, w_hbm,
                  acc_sh, srcp_v, dstp_v, srcg_v, dstg_v, dsc0, dsc1,
                  xl0, xl1, xr0, xr1, em0, em1, w0, w1, att_v, red_v,
                  sem_i, sem_g0, sem_g1, sem_sc0, sem_sc1, sem_w0, sem_w1):
    c = lax.axis_index("c")
    s = lax.axis_index("s")
    cN = c * N
    cE = c * E
    XL = (xl0, xl1)
    XR = (xr0, xr1)
    EM = (em0, em1)
    WV = (w0, w1)
    DSC = (dsc0, dsc1)
    SG = (sem_g0, sem_g1)
    SSC = (sem_sc0, sem_sc1)
    SW = (sem_w0, sem_w1)

    pltpu.sync_copy(z192_hbm, xl0)
    for r in range(TPS // SB):
        pltpu.sync_copy(xl0, acc_sh.at[pl.ds(s * TPS + r * SB, SB)])
    pltpu.sync_copy(xl0.at[pl.ds(0, TPS % SB)],
                    acc_sh.at[pl.ds(s * TPS + (TPS // SB) * SB, TPS % SB)])
    pltpu.sync_copy(att_hbm.at[c, 0], att_v)
    plsc.subcore_barrier()
    ebase = s * EPT
    iota = lax.iota(jnp.int32, 16)

    def issue_gathers(k, h, off):
        base_k = ebase + k * SB
        pltpu.async_copy(xl_hbm.at[srcg_v.at[pl.ds(off, SB)]], XL[h], SG[h])
        pltpu.async_copy(xr_hbm.at[dstg_v.at[pl.ds(off, SB)]], XR[h], SG[h])
        pltpu.async_copy(em_hbm.at[pl.ds(cE + base_k, SB)], EM[h], SG[h])
        pltpu.async_copy(dst_hbm.at[pl.ds(base_k, SB)], DSC[h], SG[h])

    def wait_gathers(h):
        pltpu.make_async_copy(
            xl_hbm.at[srcg_v.at[pl.ds(0, SB)]], XL[h], SG[h]).wait()
        pltpu.make_async_copy(
            xr_hbm.at[dstg_v.at[pl.ds(0, SB)]], XR[h], SG[h]).wait()
        pltpu.make_async_copy(em_hbm.at[pl.ds(cE, SB)], EM[h], SG[h]).wait()
        pltpu.make_async_copy(dst_hbm.at[pl.ds(0, SB)], DSC[h], SG[h]).wait()

    def drain_out(h):
        pltpu.make_async_copy(XL[h], acc_sh.at[DSC[h]], SSC[h]).wait()
        pltpu.make_async_copy(WV[h], w_hbm.at[pl.ds(cE, SB)], SW[h]).wait()

    def adjust(r16):
        srcg_v[pl.ds(r16, 16)] = srcp_v[pl.ds(r16, 16)] + cN
        dstg_v[pl.ds(r16, 16)] = dstp_v[pl.ds(r16, 16)] + cN

    def compute(h):
        xl_v, xr_v, em_v, w_v = XL[h], XR[h], EM[h], WV[h]

        def edge(i, carry2):
            irow = iota * 0 + i
            ws = []
            for u in range(3):
                acc = None
                for q in range(4):
                    j = u * 4 + q
                    t = (xl_v[i, pl.ds(j * 16, 16)]
                         + xr_v[i, pl.ds(j * 16, 16)]
                         + em_v[i, pl.ds(j * 16, 16)])
                    t = jnp.maximum(t, 0.2 * t)
                    t = t * att_v[pl.ds(j * 16, 16)]
                    acc = t if acc is None else acc + t
                for k in (8, 4, 2, 1):
                    red_v[pl.ds(u * 16, 16)] = acc
                    acc = acc + plsc.load_gather(
                        red_v, [u * 16 + jnp.bitwise_xor(iota, k)])
                lself_u = plsc.load_gather(
                    xr_v, [irow, jnp.full((16,), H2 + u, jnp.int32)])
                ws.append(jnp.exp(acc - lself_u))
            w_v[i, pl.ds(0, 16)] = jnp.where(
                iota == 0, ws[0], jnp.where(iota == 1, ws[1],
                                            jnp.where(iota == 2, ws[2], 0.0)))
            for u in range(3):
                for q in range(4):
                    j = u * 4 + q
                    xl_v[i, pl.ds(j * 16, 16)] = (
                        xl_v[i, pl.ds(j * 16, 16)] * ws[u])
            return carry2

        lax.fori_loop(0, SB, edge, 0)

    def issue_out(k, h):
        base_k = ebase + k * SB
        pltpu.async_copy(XL[h], acc_sh.at[DSC[h]], SSC[h], add=True)
        pltpu.async_copy(WV[h], w_hbm.at[pl.ds(cE + base_k, SB)], SW[h])

    pltpu.sync_copy(src_hbm.at[pl.ds(ebase, 16)], srcp_v.at[pl.ds(0, 16)])
    pltpu.sync_copy(dst_hbm.at[pl.ds(ebase, 16)], dstp_v.at[pl.ds(0, 16)])
    adjust(0)
    issue_gathers(0, 0, 0)

    def pair(g, carry):
        r16 = (g % 2) * 16
        r16n = ((g + 1) % 2) * 16
        for pp in (0, 1):
            k = 2 * g + pp
            wait_gathers(pp)
            if pp == 0:
                @pl.when(g < NP - 1)
                def _():
                    bn = ebase + (g + 1) * 16
                    pltpu.async_copy(src_hbm.at[pl.ds(bn, 16)],
                                     srcp_v.at[pl.ds(r16n, 16)], sem_i)
                    pltpu.async_copy(dst_hbm.at[pl.ds(bn, 16)],
                                     dstp_v.at[pl.ds(r16n, 16)], sem_i)

                @pl.when(g > 0)
                def _():
                    drain_out(1)
                issue_gathers(k + 1, 1, r16 + 8)
            else:
                @pl.when(g < NP - 1)
                def _():
                    pltpu.make_async_copy(
                        src_hbm.at[pl.ds(ebase, 16)],
                        srcp_v.at[pl.ds(0, 16)], sem_i).wait()
                    pltpu.make_async_copy(
                        dst_hbm.at[pl.ds(ebase, 16)],
                        dstp_v.at[pl.ds(0, 16)], sem_i).wait()
                    adjust(r16n)
                drain_out(0)

                @pl.when(g < NP - 1)
                def _():
                    issue_gathers(k + 1, 0, r16n)
            compute(pp)
            issue_out(k, pp)
        return carry

    lax.fori_loop(0, NP, pair, 0)
    drain_out(1)
    plsc.subcore_barrier()
    for r in range(TPS // SB):
        rows = s * TPS + r * SB
        pltpu.sync_copy(acc_sh.at[pl.ds(rows, SB)], xl0)
        pltpu.sync_copy(xl0, out_hbm.at[pl.ds(cN + rows, SB)])
    rows = s * TPS + (TPS // SB) * SB
    pltpu.sync_copy(acc_sh.at[pl.ds(rows, TPS % SB)], xl0.at[pl.ds(0, TPS % SB)])
    pltpu.sync_copy(xl0.at[pl.ds(0, TPS % SB)],
                    out_hbm.at[pl.ds(cN + rows, TPS % SB)])


def _sc_edge(xl, xr_aug, eemb, src, dst, att2, z192):
    mesh = plsc.VectorSubcoreMesh(core_axis_name="c", subcore_axis_name="s")
    return pl.kernel(
        _sc_edge_body,
        out_type=[jax.ShapeDtypeStruct((2 * N, H2), _f32),
                  jax.ShapeDtypeStruct((2 * E, 16), _f32)],
        mesh=mesh,
        compiler_params=pltpu.CompilerParams(needs_layout_passes=False, use_tc_tiling_on_sc=False),
        scratch_types=[
            pltpu.VMEM_SHARED((N, H2), _f32),
            pltpu.VMEM((32,), jnp.int32),
            pltpu.VMEM((32,), jnp.int32),
            pltpu.VMEM((32,), jnp.int32),
            pltpu.VMEM((32,), jnp.int32),
            pltpu.VMEM((SB,), jnp.int32),
            pltpu.VMEM((SB,), jnp.int32),
            pltpu.VMEM((SB, H2), _f32),
            pltpu.VMEM((SB, H2), _f32),
            pltpu.VMEM((SB, AW), _f32),
            pltpu.VMEM((SB, AW), _f32),
            pltpu.VMEM((SB, H2), _f32),
            pltpu.VMEM((SB, H2), _f32),
            pltpu.VMEM((SB, 16), _f32),
            pltpu.VMEM((SB, 16), _f32),
            pltpu.VMEM((H2,), _f32),
            pltpu.VMEM((48,), _f32),
        ] + [pltpu.SemaphoreType.DMA] * 7,
    )(xl, xr_aug, eemb, src, dst, att2, z192)


def _sc_den_body(w_hbm, dst_hbm, z16_hbm, out_hbm,
                 den_sh, w_v, dst_v, zer_v, sem_i, sem_sc):
    c = lax.axis_index("c")
    s = lax.axis_index("s")
    pltpu.sync_copy(z16_hbm, zer_v)
    for r in range(TPS // ZR):
        pltpu.sync_copy(zer_v, den_sh.at[pl.ds(s * TPS + r * ZR, ZR)])
    plsc.subcore_barrier()
    ebase = s * EPT

    def batch(b, carry):
        @pl.when(b > 0)
        def _():
            pltpu.make_async_copy(w_v, den_sh.at[dst_v], sem_sc).wait()
        base = ebase + b * B
        d1 = pltpu.async_copy(dst_hbm.at[pl.ds(base, B)], dst_v, sem_i)
        d2 = pltpu.async_copy(w_hbm.at[pl.ds(c * E + base, B)], w_v, sem_i)
        d1.wait()
        d2.wait()
        pltpu.async_copy(w_v, den_sh.at[dst_v], sem_sc, add=True)
        return carry

    lax.fori_loop(0, EPT // B, batch, 0)
    pltpu.make_async_copy(w_v, den_sh.at[dst_v], sem_sc).wait()
    plsc.subcore_barrier()
    for r in range(TPS // ZR):
        rows = s * TPS + r * ZR
        pltpu.sync_copy(den_sh.at[pl.ds(rows, ZR)], zer_v)
        pltpu.sync_copy(zer_v, out_hbm.at[pl.ds(c * N + rows, ZR)])


def _sc_den(w, dst, z16):
    mesh = plsc.VectorSubcoreMesh(core_axis_name="c", subcore_axis_name="s")
    return pl.kernel(
        _sc_den_body,
        out_type=jax.ShapeDtypeStruct((2 * N, 16), _f32),
        mesh=mesh,
        compiler_params=pltpu.CompilerParams(needs_layout_passes=False, use_tc_tiling_on_sc=False),
        scratch_types=[
            pltpu.VMEM_SHARED((N, 16), _f32),
            pltpu.VMEM((B, 16), _f32),
            pltpu.VMEM((B,), jnp.int32),
            pltpu.VMEM((ZR, 16), _f32),
            pltpu.SemaphoreType.DMA,
            pltpu.SemaphoreType.DMA,
        ],
    )(w, dst, z16)


def _lself_cols(xl, xr, el, att_row):
    t = xl + xr + el
    t = jnp.maximum(t, 0.2 * t)
    ta = t * att_row[None, :]
    ls = [jnp.sum(ta[:, u * HID:(u + 1) * HID], axis=1, keepdims=True)
          for u in range(3)]
    pad = jnp.zeros((xl.shape[0], 13), _f32)
    return jnp.concatenate([xr] + ls + [pad], axis=1)


def _loop_attr(laa, lab):
    la = laa + lab
    cnt = jnp.maximum(la[:, 4:5], 1.0)
    return la[:, 0:4] / cnt


def _tc_b1_body(x_r, ntm_r, laa_r, lab_r, W4_r, b4_r, Wl_r, bl_r, Wr_r, br_r,
                We_r, att_r, xl_o, xr_o):
    xb = x_r[...]
    ntm = ntm_r[...]
    h = jnp.zeros((R, HID), _f32)
    for t in range(4):
        proj = jnp.dot(xb, W4_r[t], preferred_element_type=_f32) + b4_r[t]
        h = jnp.where(ntm == t, proj, h)
    h = jnp.maximum(h, 0.0)
    xl = jnp.dot(h, Wl_r[0], preferred_element_type=_f32) + bl_r[0, 0]
    xr = jnp.dot(h, Wr_r[0], preferred_element_type=_f32) + br_r[0, 0]
    el = jnp.dot(_loop_attr(laa_r[...], lab_r[...]), We_r[0],
                 preferred_element_type=_f32)
    xl_o[...] = xl
    xr_o[...] = _lself_cols(xl, xr, el, att_r[0, 0])


def _tc_b1(x, ntm3, la, W4, b4, Wl, bl2, Wr, br2, We, att2):
    half = lambda c, i: (c * NB + i, 0)
    row = lambda c, i: (i, 0)
    return pl.pallas_call(
        _tc_b1_body,
        grid=(2, NB),
        in_specs=[
            pl.BlockSpec((R, IN_DIM), row),
            pl.BlockSpec((R, 1), lambda c, i: (i, 0)),
            pl.BlockSpec((R, 16), lambda c, i: (i, 0)),
            pl.BlockSpec((R, 16), lambda c, i: (NB + i, 0)),
            pl.BlockSpec((4, IN_DIM, HID), lambda c, i: (0, 0, 0)),
            pl.BlockSpec((4, HID), lambda c, i: (0, 0)),
            pl.BlockSpec((1, HID, H2), lambda c, i: (c, 0, 0)),
            pl.BlockSpec((1, 1, H2), lambda c, i: (c, 0, 0)),
            pl.BlockSpec((1, HID, H2), lambda c, i: (c, 0, 0)),
            pl.BlockSpec((1, 1, H2), lambda c, i: (c, 0, 0)),
            pl.BlockSpec((1, 4, H2), lambda c, i: (c, 0, 0)),
            pl.BlockSpec((1, 1, H2), lambda c, i: (c, 0, 0)),
        ],
        out_specs=[pl.BlockSpec((R, H2), half), pl.BlockSpec((R, AW), half)],
        out_shape=[jax.ShapeDtypeStruct((2 * N, H2), _f32),
                   jax.ShapeDtypeStruct((2 * N, AW), _f32)],
    )(x, ntm3, la, la, W4, b4, Wl, bl2, Wr, br2, We, att2)


def _combine_h(acca, accb, dena, denb, xla, xlb, bias2):
    chunks = []
    for c in range(2):
        acc = acca if c == 0 else accb
        den = dena if c == 0 else denb
        xlh = xla if c == 0 else xlb
        num = acc + xlh
        for u in range(3):
            d = den[:, u:u + 1] + 1.0
            hc = num[:, u * HID:(u + 1) * HID] / d + bias2[c, u * HID:(u + 1) * HID]
            chunks.append(jnp.maximum(hc, 0.0))
    return jnp.concatenate(chunks, axis=1)


def _tc_b2_body(acca_r, accb_r, dena_r, denb_r, xla_r, xlb_r, bias_r,
                laa_r, lab_r,
                Wl_r, bl_r, Wr_r, br_r, We_r, att_r, xl_o, xr_o):
    h = _combine_h(acca_r[...], accb_r[...], dena_r[...], denb_r[...],
                   xla_r[...], xlb_r[...], bias_r[:, 0, :])
    xl = jnp.dot(h, Wl_r[0], preferred_element_type=_f32) + bl_r[0, 0]
    xr = jnp.dot(h, Wr_r[0], preferred_element_type=_f32) + br_r[0, 0]
    el = jnp.dot(_loop_attr(laa_r[...], lab_r[...]), We_r[0],
                 preferred_element_type=_f32)
    xl_o[...] = xl
    xr_o[...] = _lself_cols(xl, xr, el, att_r[0, 0])


def _tc_b2(acc1, den1, xl1, bias1_2, la, Wl, bl2, Wr, br2, We, att2):
    half = lambda c, i: (c * NB + i, 0)
    return pl.pallas_call(
        _tc_b2_body,
        grid=(2, NB),
        in_specs=[
            pl.BlockSpec((R, H2), lambda c, i: (i, 0)),
            pl.BlockSpec((R, H2), lambda c, i: (NB + i, 0)),
            pl.BlockSpec((R, 16), lambda c, i: (i, 0)),
            pl.BlockSpec((R, 16), lambda c, i: (NB + i, 0)),
            pl.BlockSpec((R, H2), lambda c, i: (i, 0)),
            pl.BlockSpec((R, H2), lambda c, i: (NB + i, 0)),
            pl.BlockSpec((2, 1, H2), lambda c, i: (0, 0, 0)),
            pl.BlockSpec((R, 16), lambda c, i: (i, 0)),
            pl.BlockSpec((R, 16), lambda c, i: (NB + i, 0)),
            pl.BlockSpec((1, HC, H2), lambda c, i: (c, 0, 0)),
            pl.BlockSpec((1, 1, H2), lambda c, i: (c, 0, 0)),
            pl.BlockSpec((1, HC, H2), lambda c, i: (c, 0, 0)),
            pl.BlockSpec((1, 1, H2), lambda c, i: (c, 0, 0)),
            pl.BlockSpec((1, 4, H2), lambda c, i: (c, 0, 0)),
            pl.BlockSpec((1, 1, H2), lambda c, i: (c, 0, 0)),
        ],
        out_specs=[pl.BlockSpec((R, H2), half), pl.BlockSpec((R, AW), half)],
        out_shape=[jax.ShapeDtypeStruct((2 * N, H2), _f32),
                   jax.ShapeDtypeStruct((2 * N, AW), _f32)],
    )(acc1, acc1, den1, den1, xl1, xl1, bias1_2, la, la,
      Wl, bl2, Wr, br2, We, att2)


def _tc_eemb_body(ea_r, We_r, out_o):
    out_o[...] = jnp.dot(ea_r[...], We_r[0], preferred_element_type=_f32)


def _tc_eemb(ea, We):
    return pl.pallas_call(
        _tc_eemb_body,
        grid=(2, NEB),
        in_specs=[
            pl.BlockSpec((EB, 4), lambda c, i: (i, 0)),
            pl.BlockSpec((1, 4, H2), lambda c, i: (c, 0, 0)),
        ],
        out_specs=pl.BlockSpec((EB, H2), lambda c, i: (c * NEB + i, 0)),
        out_shape=jax.ShapeDtypeStruct((2 * E, H2), _f32),
    )(ea, We)


def _tc_pool_body(acca_r, accb_r, dena_r, denb_r, xla_r, xlb_r, bias_r,
                  bat_r, Wo_r, bo_r, out_o, sums, cnts):
    i = pl.program_id(0)

    @pl.when(i == 0)
    def _():
        sums[...] = jnp.zeros((NUM_GRAPHS, HC), _f32)
        cnts[...] = jnp.zeros((NUM_GRAPHS, 128), _f32)

    h = _combine_h(acca_r[...], accb_r[...], dena_r[...], denb_r[...],
                   xla_r[...], xlb_r[...], bias_r[:, 0, :])
    bat = bat_r[0]
    gid = lax.broadcasted_iota(jnp.int32, (NUM_GRAPHS, R), 0)
    onehot = (bat == gid).astype(_f32)
    sums[...] += jnp.dot(onehot, h, preferred_element_type=_f32)
    cnts[...] += jnp.dot(onehot, jnp.ones((R, 128), _f32),
                         preferred_element_type=_f32)

    @pl.when(i == NB - 1)
    def _():
        pooled = sums[...] / jnp.maximum(cnts[:, 0:1], 1.0)
        out_o[...] = jnp.tanh(
            jnp.dot(pooled, Wo_r[...], preferred_element_type=_f32) + bo_r[0])


def _tc_pool(acc2, den2, xl2, bias2_2, bat3, Wo, bo2):
    return pl.pallas_call(
        _tc_pool_body,
        grid=(NB,),
        in_specs=[
            pl.BlockSpec((R, H2), lambda i: (i, 0)),
            pl.BlockSpec((R, H2), lambda i: (NB + i, 0)),
            pl.BlockSpec((R, 16), lambda i: (i, 0)),
            pl.BlockSpec((R, 16), lambda i: (NB + i, 0)),
            pl.BlockSpec((R, H2), lambda i: (i, 0)),
            pl.BlockSpec((R, H2), lambda i: (NB + i, 0)),
            pl.BlockSpec((2, 1, H2), lambda i: (0, 0, 0)),
            pl.BlockSpec((1, 1, R), lambda i: (i, 0, 0)),
            pl.BlockSpec((HC, OUT_DIM), lambda i: (0, 0)),
            pl.BlockSpec((1, OUT_DIM), lambda i: (0, 0)),
        ],
        out_specs=pl.BlockSpec((NUM_GRAPHS, OUT_DIM), lambda i: (0, 0)),
        out_shape=jax.ShapeDtypeStruct((NUM_GRAPHS, OUT_DIM), _f32),
        scratch_shapes=[pltpu.VMEM((NUM_GRAPHS, HC), _f32),
                        pltpu.VMEM((NUM_GRAPHS, 128), _f32)],
    )(acc2, acc2, den2, den2, xl2, xl2, bias2_2, bat3, Wo, bo2)


def kernel(x, edge_index, edge_attr, node_type_mask, batch, params):
    src = edge_index[0].astype(jnp.int32)
    dst = edge_index[1].astype(jnp.int32)
    ea_pad = jnp.concatenate(
        [edge_attr, jnp.ones((E, 1), _f32), jnp.zeros((E, 11), _f32)], axis=1)
    ntm3 = node_type_mask.astype(jnp.int32).reshape(N, 1)
    bat3 = batch.astype(jnp.int32).reshape(NB, 1, R)
    z16 = jnp.zeros((ZR, 16), _f32)
    z192 = jnp.zeros((8, H2), _f32)

    p = params
    W4 = jnp.stack([p["W_joint"], p["W_obj"], p["W_tcp"], p["W_goal"]])
    b4 = jnp.stack([p["b_joint"], p["b_obj"], p["b_tcp"], p["b_goal"]])
    c1, c2 = p["convs"][0], p["convs"][1]

    la = _sc_loopattr(ea_pad, dst, z16)

    att1 = c1["att"].reshape(2, 1, H2)
    xl1, xr1 = _tc_b1(x, ntm3, la, W4, b4,
                      _halves(c1["Wl"]), c1["bl"].reshape(2, 1, H2),
                      _halves(c1["Wr"]), c1["br"].reshape(2, 1, H2),
                      _halves(c1["We"]), att1)
    em1 = _tc_eemb(edge_attr, _halves(c1["We"]))
    acc1, w1 = _sc_edge(xl1, xr1, em1, src, dst, att1, z192)
    den1 = _sc_den(w1, dst, z16)

    att2 = c2["att"].reshape(2, 1, H2)
    xl2, xr2 = _tc_b2(acc1, den1, xl1, c1["bias"].reshape(2, 1, H2), la,
                      _halves(c2["Wl"]), c2["bl"].reshape(2, 1, H2),
                      _halves(c2["Wr"]), c2["br"].reshape(2, 1, H2),
                      _halves(c2["We"]), att2)
    em2 = _tc_eemb(edge_attr, _halves(c2["We"]))
    acc2, w2 = _sc_edge(xl2, xr2, em2, src, dst, att2, z192)
    den2 = _sc_den(w2, dst, z16)

    return _tc_pool(acc2, den2, xl2, c2["bias"].reshape(2, 1, H2), bat3,
                    p["W_out"], p["b_out"].reshape(1, OUT_DIM))


# speedup vs baseline: 11.0024x; 1.0002x over previous
import jax
import jax.numpy as jnp
from jax import lax
from jax.experimental import pallas as pl
from jax.experimental.pallas import tpu as pltpu
from jax.experimental.pallas import tpu_sc as plsc

N = 10000
E = 160000
IN_DIM = 128
HID = 64
HEADS = 6
HC = HID * HEADS
OUT_DIM = 32
NUM_GRAPHS = 64

H2 = HC // 2
AW = H2 + 16
R = 1000
NB = N // R
EB = 2000
NEB = E // EB
TPS = N // 16
ZR = 125
B = 80
EPT = E // 16
B0 = 40

_f32 = jnp.float32


def _halves(W):
    return W.reshape(W.shape[0], 2, H2).transpose(1, 0, 2)


def _sc_loopattr_body(ea_hbm, dst_hbm, z16_hbm, out_hbm,
                      la_sh, ea_v, dst_v, zer_v, sem_i, sem_sc):
    c = lax.axis_index("c")
    s = lax.axis_index("s")
    pltpu.sync_copy(z16_hbm, zer_v)
    for r in range(TPS // ZR):
        pltpu.sync_copy(zer_v, la_sh.at[pl.ds(s * TPS + r * ZR, ZR)])
    plsc.subcore_barrier()
    wid = s * 2 + c
    ebase = wid * (E // 32)

    def batch(b, carry):
        @pl.when(b > 0)
        def _():
            pltpu.make_async_copy(ea_v, la_sh.at[dst_v], sem_sc).wait()
        base = ebase + b * B0
        d1 = pltpu.async_copy(dst_hbm.at[pl.ds(base, B0)], dst_v, sem_i)
        d2 = pltpu.async_copy(ea_hbm.at[pl.ds(base, B0)], ea_v, sem_i)
        d1.wait()
        d2.wait()
        pltpu.async_copy(ea_v, la_sh.at[dst_v], sem_sc, add=True)
        return carry

    lax.fori_loop(0, (E // 32) // B0, batch, 0)
    pltpu.make_async_copy(ea_v, la_sh.at[dst_v], sem_sc).wait()
    plsc.subcore_barrier()
    for r in range(TPS // ZR):
        rows = s * TPS + r * ZR
        pltpu.sync_copy(la_sh.at[pl.ds(rows, ZR)], zer_v)
        pltpu.sync_copy(zer_v, out_hbm.at[pl.ds(c * N + rows, ZR)])


def _sc_loopattr(ea_pad, dst, z16):
    mesh = plsc.VectorSubcoreMesh(core_axis_name="c", subcore_axis_name="s")
    return pl.kernel(
        _sc_loopattr_body,
        out_type=jax.ShapeDtypeStruct((2 * N, 16), _f32),
        mesh=mesh,
        compiler_params=pltpu.CompilerParams(needs_layout_passes=False, use_tc_tiling_on_sc=False),
        scratch_types=[
            pltpu.VMEM_SHARED((N, 16), _f32),
            pltpu.VMEM((B0, 16), _f32),
            pltpu.VMEM((B0,), jnp.int32),
            pltpu.VMEM((ZR, 16), _f32),
            pltpu.SemaphoreType.DMA,
            pltpu.SemaphoreType.DMA,
        ],
    )(ea_pad, dst, z16)


SB = 8
NSB = EPT // SB
NP = NSB // 2


def _sc_edge_body(xl_hbm, xr_hbm, em_hbm, src_hbm, dst_hbm, att_hbm, z192_hbm,
                  out_hbm, w_hbm,
                  acc_sh, srcp_v, dstp_v, srcg_v, dstg_v, dsc0, dsc1,
                  xl0, xl1, xr0, xr1, em0, em1, w0, w1, att_v, red_v,
                  sem_i, sem_g0, sem_g1, sem_sc0, sem_sc1, sem_w0, sem_w1):
    c = lax.axis_index("c")
    s = lax.axis_index("s")
    cN = c * N
    cE = c * E
    XL = (xl0, xl1)
    XR = (xr0, xr1)
    EM = (em0, em1)
    WV = (w0, w1)
    DSC = (dsc0, dsc1)
    SG = (sem_g0, sem_g1)
    SSC = (sem_sc0, sem_sc1)
    SW = (sem_w0, sem_w1)

    pltpu.sync_copy(z192_hbm, xl0)
    for r in range(TPS // SB):
        pltpu.sync_copy(xl0, acc_sh.at[pl.ds(s * TPS + r * SB, SB)])
    pltpu.sync_copy(xl0.at[pl.ds(0, TPS % SB)],
                    acc_sh.at[pl.ds(s * TPS + (TPS // SB) * SB, TPS % SB)])
    pltpu.sync_copy(att_hbm.at[c, 0], att_v)
    plsc.subcore_barrier()
    ebase = s * EPT
    iota = lax.iota(jnp.int32, 16)

    def issue_gathers(k, h, off):
        base_k = ebase + k * SB
        pltpu.async_copy(xl_hbm.at[srcg_v.at[pl.ds(off, SB)]], XL[h], SG[h])
        pltpu.async_copy(xr_hbm.at[dstg_v.at[pl.ds(off, SB)]], XR[h], SG[h])
        pltpu.async_copy(em_hbm.at[pl.ds(cE + base_k, SB)], EM[h], SG[h])
        pltpu.async_copy(dst_hbm.at[pl.ds(base_k, SB)], DSC[h], SG[h])

    def wait_gathers(h):
        pltpu.make_async_copy(
            xl_hbm.at[srcg_v.at[pl.ds(0, SB)]], XL[h], SG[h]).wait()
        pltpu.make_async_copy(
            xr_hbm.at[dstg_v.at[pl.ds(0, SB)]], XR[h], SG[h]).wait()
        pltpu.make_async_copy(em_hbm.at[pl.ds(cE, SB)], EM[h], SG[h]).wait()
        pltpu.make_async_copy(dst_hbm.at[pl.ds(0, SB)], DSC[h], SG[h]).wait()

    def drain_out(h):
        pltpu.make_async_copy(XL[h], acc_sh.at[DSC[h]], SSC[h]).wait()
        pltpu.make_async_copy(WV[h], w_hbm.at[pl.ds(cE, SB)], SW[h]).wait()

    def adjust(r16):
        srcg_v[pl.ds(r16, 16)] = srcp_v[pl.ds(r16, 16)] + cN
        dstg_v[pl.ds(r16, 16)] = dstp_v[pl.ds(r16, 16)] + cN

    def compute(h):
        xl_v, xr_v, em_v, w_v = XL[h], XR[h], EM[h], WV[h]

        def edge(i, carry2):
            irow = iota * 0 + i
            ws = []
            for u in range(3):
                acc = None
                for q in range(4):
                    j = u * 4 + q
                    t = (xl_v[i, pl.ds(j * 16, 16)]
                         + xr_v[i, pl.ds(j * 16, 16)]
                         + em_v[i, pl.ds(j * 16, 16)])
                    t = jnp.maximum(t, 0.2 * t)
                    t = t * att_v[pl.ds(j * 16, 16)]
                    acc = t if acc is None else acc + t
                for k in (8, 4, 2, 1):
                    red_v[pl.ds(u * 16, 16)] = acc
                    acc = acc + plsc.load_gather(
                        red_v, [u * 16 + jnp.bitwise_xor(iota, k)])
                lself_u = plsc.load_gather(
                    xr_v, [irow, jnp.full((16,), H2 + u, jnp.int32)])
                ws.append(jnp.exp(acc - lself_u))
            w_v[i, pl.ds(0, 16)] = jnp.where(
                iota == 0, ws[0], jnp.where(iota == 1, ws[1],
                                            jnp.where(iota == 2, ws[2], 0.0)))
            for u in range(3):
                for q in range(4):
                    j = u * 4 + q
                    xl_v[i, pl.ds(j * 16, 16)] = (
                        xl_v[i, pl.ds(j * 16, 16)] * ws[u])
            return carry2

        lax.fori_loop(0, SB, edge, 0)

    def issue_out(k, h):
        base_k = ebase + k * SB
        pltpu.async_copy(XL[h], acc_sh.at[DSC[h]], SSC[h], add=True)
        pltpu.async_copy(WV[h], w_hbm.at[pl.ds(cE + base_k, SB)], SW[h])

    pltpu.sync_copy(src_hbm.at[pl.ds(ebase, 16)], srcp_v.at[pl.ds(0, 16)])
    pltpu.sync_copy(dst_hbm.at[pl.ds(ebase, 16)], dstp_v.at[pl.ds(0, 16)])
    adjust(0)
    issue_gathers(0, 0, 0)

    def pair(g, carry):
        r16 = (g % 2) * 16
        r16n = ((g + 1) % 2) * 16
        for pp in (0, 1):
            k = 2 * g + pp
            wait_gathers(pp)
            if pp == 0:
                @pl.when(g < NP - 1)
                def _():
                    bn = ebase + (g + 1) * 16
                    pltpu.async_copy(src_hbm.at[pl.ds(bn, 16)],
                                     srcp_v.at[pl.ds(r16n, 16)], sem_i)
                    pltpu.async_copy(dst_hbm.at[pl.ds(bn, 16)],
                                     dstp_v.at[pl.ds(r16n, 16)], sem_i)

                @pl.when(g > 0)
                def _():
                    drain_out(1)
                issue_gathers(k + 1, 1, r16 + 8)
            else:
                @pl.when(g < NP - 1)
                def _():
                    pltpu.make_async_copy(
                        src_hbm.at[pl.ds(ebase, 16)],
                        srcp_v.at[pl.ds(0, 16)], sem_i).wait()
                    pltpu.make_async_copy(
                        dst_hbm.at[pl.ds(ebase, 16)],
                        dstp_v.at[pl.ds(0, 16)], sem_i).wait()
                    adjust(r16n)
                drain_out(0)

                @pl.when(g < NP - 1)
                def _():
                    issue_gathers(k + 1, 0, r16n)
            compute(pp)
            issue_out(k, pp)
        return carry

    lax.fori_loop(0, NP, pair, 0)
    drain_out(1)
    plsc.subcore_barrier()
    for r in range(TPS // SB):
        rows = s * TPS + r * SB
        pltpu.sync_copy(acc_sh.at[pl.ds(rows, SB)], xl0)
        pltpu.sync_copy(xl0, out_hbm.at[pl.ds(cN + rows, SB)])
    rows = s * TPS + (TPS // SB) * SB
    pltpu.sync_copy(acc_sh.at[pl.ds(rows, TPS % SB)], xl0.at[pl.ds(0, TPS % SB)])
    pltpu.sync_copy(xl0.at[pl.ds(0, TPS % SB)],
                    out_hbm.at[pl.ds(cN + rows, TPS % SB)])


def _sc_edge(xl, xr_aug, eemb, src, dst, att2, z192):
    mesh = plsc.VectorSubcoreMesh(core_axis_name="c", subcore_axis_name="s")
    return pl.kernel(
        _sc_edge_body,
        out_type=[jax.ShapeDtypeStruct((2 * N, H2), _f32),
                  jax.ShapeDtypeStruct((2 * E, 16), _f32)],
        mesh=mesh,
        compiler_params=pltpu.CompilerParams(needs_layout_passes=False, use_tc_tiling_on_sc=False),
        scratch_types=[
            pltpu.VMEM_SHARED((N, H2), _f32),
            pltpu.VMEM((32,), jnp.int32),
            pltpu.VMEM((32,), jnp.int32),
            pltpu.VMEM((32,), jnp.int32),
            pltpu.VMEM((32,), jnp.int32),
            pltpu.VMEM((SB,), jnp.int32),
            pltpu.VMEM((SB,), jnp.int32),
            pltpu.VMEM((SB, H2), _f32),
            pltpu.VMEM((SB, H2), _f32),
            pltpu.VMEM((SB, AW), _f32),
            pltpu.VMEM((SB, AW), _f32),
            pltpu.VMEM((SB, H2), _f32),
            pltpu.VMEM((SB, H2), _f32),
            pltpu.VMEM((SB, 16), _f32),
            pltpu.VMEM((SB, 16), _f32),
            pltpu.VMEM((H2,), _f32),
            pltpu.VMEM((48,), _f32),
        ] + [pltpu.SemaphoreType.DMA] * 7,
    )(xl, xr_aug, eemb, src, dst, att2, z192)


def _sc_den_body(w_hbm, dst_hbm, z16_hbm, out_hbm,
                 den_sh, w_v, dst_v, zer_v, sem_i, sem_sc):
    c = lax.axis_index("c")
    s = lax.axis_index("s")
    pltpu.sync_copy(z16_hbm, zer_v)
    for r in range(TPS // ZR):
        pltpu.sync_copy(zer_v, den_sh.at[pl.ds(s * TPS + r * ZR, ZR)])
    plsc.subcore_barrier()
    ebase = s * EPT

    def batch(b, carry):
        @pl.when(b > 0)
        def _():
            pltpu.make_async_copy(w_v, den_sh.at[dst_v], sem_sc).wait()
        base = ebase + b * B
        d1 = pltpu.async_copy(dst_hbm.at[pl.ds(base, B)], dst_v, sem_i)
        d2 = pltpu.async_copy(w_hbm.at[pl.ds(c * E + base, B)], w_v, sem_i)
        d1.wait()
        d2.wait()
        pltpu.async_copy(w_v, den_sh.at[dst_v], sem_sc, add=True)
        return carry

    lax.fori_loop(0, EPT // B, batch, 0)
    pltpu.make_async_copy(w_v, den_sh.at[dst_v], sem_sc).wait()
    plsc.subcore_barrier()
    for r in range(TPS // ZR):
        rows = s * TPS + r * ZR
        pltpu.sync_copy(den_sh.at[pl.ds(rows, ZR)], zer_v)
        pltpu.sync_copy(zer_v, out_hbm.at[pl.ds(c * N + rows, ZR)])


def _sc_den(w, dst, z16):
    mesh = plsc.VectorSubcoreMesh(core_axis_name="c", subcore_axis_name="s")
    return pl.kernel(
        _sc_den_body,
        out_type=jax.ShapeDtypeStruct((2 * N, 16), _f32),
        mesh=mesh,
        compiler_params=pltpu.CompilerParams(needs_layout_passes=False, use_tc_tiling_on_sc=False),
        scratch_types=[
            pltpu.VMEM_SHARED((N, 16), _f32),
            pltpu.VMEM((B, 16), _f32),
            pltpu.VMEM((B,), jnp.int32),
            pltpu.VMEM((ZR, 16), _f32),
            pltpu.SemaphoreType.DMA,
            pltpu.SemaphoreType.DMA,
        ],
    )(w, dst, z16)


def _lself_cols(xl, xr, el, att_row):
    t = xl + xr + el
    t = jnp.maximum(t, 0.2 * t)
    ta = t * att_row[None, :]
    ls = [jnp.sum(ta[:, u * HID:(u + 1) * HID], axis=1, keepdims=True)
          for u in range(3)]
    pad = jnp.zeros((xl.shape[0], 13), _f32)
    return jnp.concatenate([xr] + ls + [pad], axis=1)


def _loop_attr(laa, lab):
    la = laa + lab
    cnt = jnp.maximum(la[:, 4:5], 1.0)
    return la[:, 0:4] / cnt


def _tc_b1_body(x_r, ntm_r, laa_r, lab_r, W4_r, b4_r, Wl_r, bl_r, Wr_r, br_r,
                We_r, att_r, xl_o, xr_o):
    xb = x_r[...]
    ntm = ntm_r[...]
    h = jnp.zeros((R, HID), _f32)
    for t in range(4):
        proj = jnp.dot(xb, W4_r[t], preferred_element_type=_f32) + b4_r[t]
        h = jnp.where(ntm == t, proj, h)
    h = jnp.maximum(h, 0.0)
    xl = jnp.dot(h, Wl_r[0], preferred_element_type=_f32) + bl_r[0, 0]
    xr = jnp.dot(h, Wr_r[0], preferred_element_type=_f32) + br_r[0, 0]
    el = jnp.dot(_loop_attr(laa_r[...], lab_r[...]), We_r[0],
                 preferred_element_type=_f32)
    xl_o[...] = xl
    xr_o[...] = _lself_cols(xl, xr, el, att_r[0, 0])


def _tc_b1(x, ntm3, la, W4, b4, Wl, bl2, Wr, br2, We, att2):
    half = lambda c, i: (c * NB + i, 0)
    row = lambda c, i: (i, 0)
    return pl.pallas_call(
        _tc_b1_body,
        grid=(2, NB),
        in_specs=[
            pl.BlockSpec((R, IN_DIM), row),
            pl.BlockSpec((R, 1), lambda c, i: (i, 0)),
            pl.BlockSpec((R, 16), lambda c, i: (i, 0)),
            pl.BlockSpec((R, 16), lambda c, i: (NB + i, 0)),
            pl.BlockSpec((4, IN_DIM, HID), lambda c, i: (0, 0, 0)),
            pl.BlockSpec((4, HID), lambda c, i: (0, 0)),
            pl.BlockSpec((1, HID, H2), lambda c, i: (c, 0, 0)),
            pl.BlockSpec((1, 1, H2), lambda c, i: (c, 0, 0)),
            pl.BlockSpec((1, HID, H2), lambda c, i: (c, 0, 0)),
            pl.BlockSpec((1, 1, H2), lambda c, i: (c, 0, 0)),
            pl.BlockSpec((1, 4, H2), lambda c, i: (c, 0, 0)),
            pl.BlockSpec((1, 1, H2), lambda c, i: (c, 0, 0)),
        ],
        out_specs=[pl.BlockSpec((R, H2), half), pl.BlockSpec((R, AW), half)],
        out_shape=[jax.ShapeDtypeStruct((2 * N, H2), _f32),
                   jax.ShapeDtypeStruct((2 * N, AW), _f32)],
    )(x, ntm3, la, la, W4, b4, Wl, bl2, Wr, br2, We, att2)


def _combine_h(acca, accb, dena, denb, xla, xlb, bias2):
    chunks = []
    for c in range(2):
        acc = acca if c == 0 else accb
        den = dena if c == 0 else denb
        xlh = xla if c == 0 else xlb
        num = acc + xlh
        for u in range(3):
            d = den[:, u:u + 1] + 1.0
            hc = num[:, u * HID:(u + 1) * HID] / d + bias2[c, u * HID:(u + 1) * HID]
            chunks.append(jnp.maximum(hc, 0.0))
    return jnp.concatenate(chunks, axis=1)


def _tc_b2_body(acca_r, accb_r, dena_r, denb_r, xla_r, xlb_r, bias_r,
                laa_r, lab_r,
                Wl_r, bl_r, Wr_r, br_r, We_r, att_r, xl_o, xr_o):
    h = _combine_h(acca_r[...], accb_r[...], dena_r[...], denb_r[...],
                   xla_r[...], xlb_r[...], bias_r[:, 0, :])
    xl = jnp.dot(h, Wl_r[0], preferred_element_type=_f32) + bl_r[0, 0]
    xr = jnp.dot(h, Wr_r[0], preferred_element_type=_f32) + br_r[0, 0]
    el = jnp.dot(_loop_attr(laa_r[...], lab_r[...]), We_r[0],
                 preferred_element_type=_f32)
    xl_o[...] = xl
    xr_o[...] = _lself_cols(xl, xr, el, att_r[0, 0])


def _tc_b2(acc1, den1, xl1, bias1_2, la, Wl, bl2, Wr, br2, We, att2):
    half = lambda c, i: (c * NB + i, 0)
    return pl.pallas_call(
        _tc_b2_body,
        grid=(2, NB),
        in_specs=[
            pl.BlockSpec((R, H2), lambda c, i: (i, 0)),
            pl.BlockSpec((R, H2), lambda c, i: (NB + i, 0)),
            pl.BlockSpec((R, 16), lambda c, i: (i, 0)),
            pl.BlockSpec((R, 16), lambda c, i: (NB + i, 0)),
            pl.BlockSpec((R, H2), lambda c, i: (i, 0)),
            pl.BlockSpec((R, H2), lambda c, i: (NB + i, 0)),
            pl.BlockSpec((2, 1, H2), lambda c, i: (0, 0, 0)),
            pl.BlockSpec((R, 16), lambda c, i: (i, 0)),
            pl.BlockSpec((R, 16), lambda c, i: (NB + i, 0)),
            pl.BlockSpec((1, HC, H2), lambda c, i: (c, 0, 0)),
            pl.BlockSpec((1, 1, H2), lambda c, i: (c, 0, 0)),
            pl.BlockSpec((1, HC, H2), lambda c, i: (c, 0, 0)),
            pl.BlockSpec((1, 1, H2), lambda c, i: (c, 0, 0)),
            pl.BlockSpec((1, 4, H2), lambda c, i: (c, 0, 0)),
            pl.BlockSpec((1, 1, H2), lambda c, i: (c, 0, 0)),
        ],
        out_specs=[pl.BlockSpec((R, H2), half), pl.BlockSpec((R, AW), half)],
        out_shape=[jax.ShapeDtypeStruct((2 * N, H2), _f32),
                   jax.ShapeDtypeStruct((2 * N, AW), _f32)],
    )(acc1, acc1, den1, den1, xl1, xl1, bias1_2, la, la,
      Wl, bl2, Wr, br2, We, att2)


def _tc_eemb_body(ea_r, We_r, out_o):
    out_o[...] = jnp.dot(ea_r[...], We_r[0], preferred_element_type=_f32)


def _tc_eemb(ea, We):
    return pl.pallas_call(
        _tc_eemb_body,
        grid=(2, NEB),
        in_specs=[
            pl.BlockSpec((EB, 4), lambda c, i: (i, 0)),
            pl.BlockSpec((1, 4, H2), lambda c, i: (c, 0, 0)),
        ],
        out_specs=pl.BlockSpec((EB, H2), lambda c, i: (c * NEB + i, 0)),
        out_shape=jax.ShapeDtypeStruct((2 * E, H2), _f32),
    )(ea, We)


def _tc_pool_body(acca_r, accb_r, dena_r, denb_r, xla_r, xlb_r, bias_r,
                  bat_r, Wo_r, bo_r, out_o, sums, cnts):
    i = pl.program_id(0)

    @pl.when(i == 0)
    def _():
        sums[...] = jnp.zeros((NUM_GRAPHS, HC), _f32)
        cnts[...] = jnp.zeros((NUM_GRAPHS, 128), _f32)

    h = _combine_h(acca_r[...], accb_r[...], dena_r[...], denb_r[...],
                   xla_r[...], xlb_r[...], bias_r[:, 0, :])
    bat = bat_r[0]
    gid = lax.broadcasted_iota(jnp.int32, (NUM_GRAPHS, R), 0)
    onehot = (bat == gid).astype(_f32)
    sums[...] += jnp.dot(onehot, h, preferred_element_type=_f32)
    cnts[...] += jnp.dot(onehot, jnp.ones((R, 128), _f32),
                         preferred_element_type=_f32)

    @pl.when(i == NB - 1)
    def _():
        pooled = sums[...] / jnp.maximum(cnts[:, 0:1], 1.0)
        out_o[...] = jnp.tanh(
            jnp.dot(pooled, Wo_r[...], preferred_element_type=_f32) + bo_r[0])


def _tc_pool(acc2, den2, xl2, bias2_2, bat3, Wo, bo2):
    return pl.pallas_call(
        _tc_pool_body,
        grid=(NB,),
        in_specs=[
            pl.BlockSpec((R, H2), lambda i: (i, 0)),
            pl.BlockSpec((R, H2), lambda i: (NB + i, 0)),
            pl.BlockSpec((R, 16), lambda i: (i, 0)),
            pl.BlockSpec((R, 16), lambda i: (NB + i, 0)),
            pl.BlockSpec((R, H2), lambda i: (i, 0)),
            pl.BlockSpec((R, H2), lambda i: (NB + i, 0)),
            pl.BlockSpec((2, 1, H2), lambda i: (0, 0, 0)),
            pl.BlockSpec((1, 1, R), lambda i: (i, 0, 0)),
            pl.BlockSpec((HC, OUT_DIM), lambda i: (0, 0)),
            pl.BlockSpec((1, OUT_DIM), lambda i: (0, 0)),
        ],
        out_specs=pl.BlockSpec((NUM_GRAPHS, OUT_DIM), lambda i: (0, 0)),
        out_shape=jax.ShapeDtypeStruct((NUM_GRAPHS, OUT_DIM), _f32),
        scratch_shapes=[pltpu.VMEM((NUM_GRAPHS, HC), _f32),
                        pltpu.VMEM((NUM_GRAPHS, 128), _f32)],
    )(acc2, acc2, den2, den2, xl2, xl2, bias2_2, bat3, Wo, bo2)


def kernel(x, edge_index, edge_attr, node_type_mask, batch, params):
    src = edge_index[0].astype(jnp.int32)
    dst = edge_index[1].astype(jnp.int32)
    ea_pad = jnp.concatenate(
        [edge_attr, jnp.ones((E, 1), _f32), jnp.zeros((E, 11), _f32)], axis=1)
    ntm3 = node_type_mask.astype(jnp.int32).reshape(N, 1)
    bat3 = batch.astype(jnp.int32).reshape(NB, 1, R)
    z16 = jnp.zeros((ZR, 16), _f32)
    z192 = jnp.zeros((8, H2), _f32)

    p = params
    W4 = jnp.stack([p["W_joint"], p["W_obj"], p["W_tcp"], p["W_goal"]])
    b4 = jnp.stack([p["b_joint"], p["b_obj"], p["b_tcp"], p["b_goal"]])
    c1, c2 = p["convs"][0], p["convs"][1]

    la = _sc_loopattr(ea_pad, dst, z16)

    att1 = c1["att"].reshape(2, 1, H2)
    xl1, xr1 = _tc_b1(x, ntm3, la, W4, b4,
                      _halves(c1["Wl"]), c1["bl"].reshape(2, 1, H2),
                      _halves(c1["Wr"]), c1["br"].reshape(2, 1, H2),
                      _halves(c1["We"]), att1)
    em1 = _tc_eemb(edge_attr, _halves(c1["We"]))
    acc1, w1 = _sc_edge(xl1, xr1, em1, src, dst, att1, z192)
    den1 = _sc_den(w1, dst, z16)

    att2 = c2["att"].reshape(2, 1, H2)
    xl2, xr2 = _tc_b2(acc1, den1, xl1, c1["bias"].reshape(2, 1, H2), la,
                      _halves(c2["Wl"]), c2["bl"].reshape(2, 1, H2),
                      _halves(c2["Wr"]), c2["br"].reshape(2, 1, H2),
                      _halves(c2["We"]), att2)
    em2 = _tc_eemb(edge_attr, _halves(c2["We"]))
    acc2, w2 = _sc_edge(xl2, xr2, em2, src, dst, att2, z192)
    den2 = _sc_den(w2, dst, z16)

    return _tc_pool(acc2, den2, xl2, c2["bias"].reshape(2, 1, H2), bat3,
                    p["W_out"], p["b_out"].reshape(1, OUT_DIM))
